# Initial kernel scaffold; baseline (speedup 1.0000x reference)
import functools

import jax
import jax.numpy as jnp
from jax import lax
from jax.experimental import pallas as pl
from jax.experimental.pallas import tpu as pltpu
from jax.experimental.pallas import tpu_sc as plsc

N = 10000
E = 320000
C = 128
D = 128

NPAD = 10240
RCH = 80
NROW = E // RCH
EPT_A = E // 16
RA = EPT_A // RCH
EPT_B = E // 32
RB = EPT_B // RCH
BN = 2000

_MESH = dict(core_axis_name="c", subcore_axis_name="s", num_cores=2,
             num_subcores=16)


def _tc_in_body(x_ref, w_ref, p_ref, h_ref, sdt_ref):
    h = jnp.dot(x_ref[...], w_ref[...], preferred_element_type=jnp.float32)
    h_ref[...] = h
    sdt_ref[...] = lax.dot_general(p_ref[...], h, (((1,), (1,)), ((), ())),
                                   preferred_element_type=jnp.float32)


def _tc_in(x, w, p):
    return pl.pallas_call(
        _tc_in_body,
        grid=(N // BN,),
        in_specs=[pl.BlockSpec((BN, D), lambda i: (i, 0)),
                  pl.BlockSpec((D, C), lambda i: (0, 0)),
                  pl.BlockSpec((8, D), lambda i: (0, 0))],
        out_specs=[pl.BlockSpec((BN, C), lambda i: (i, 0)),
                   pl.BlockSpec((8, BN), lambda i: (0, i))],
        out_shape=[jax.ShapeDtypeStruct((N, C), jnp.float32),
                   jax.ShapeDtypeStruct((8, N), jnp.float32)],
    )(x, w, p)


def _tc_mid_body(q0_ref, q1_ref, b_ref, w_ref, p_ref, h_ref, sdt_ref):
    o = jnp.maximum(q0_ref[...] + q1_ref[...] + b_ref[...], 0.0)
    h = jnp.dot(o, w_ref[...], preferred_element_type=jnp.float32)
    h_ref[...] = h
    sdt_ref[...] = lax.dot_general(p_ref[...], h, (((1,), (1,)), ((), ())),
                                   preferred_element_type=jnp.float32)


def _tc_mid(q0, q1, b, w, p):
    return pl.pallas_call(
        _tc_mid_body,
        grid=(N // BN,),
        in_specs=[pl.BlockSpec((BN, C), lambda i: (i, 0)),
                  pl.BlockSpec((BN, C), lambda i: (i, 0)),
                  pl.BlockSpec((1, C), lambda i: (0, 0)),
                  pl.BlockSpec((C, C), lambda i: (0, 0)),
                  pl.BlockSpec((8, C), lambda i: (0, 0))],
        out_specs=[pl.BlockSpec((BN, C), lambda i: (i, 0)),
                   pl.BlockSpec((8, BN), lambda i: (0, i))],
        out_shape=[jax.ShapeDtypeStruct((N, C), jnp.float32),
                   jax.ShapeDtypeStruct((8, N), jnp.float32)],
    )(q0, q1, b, w, p)


def _tc_out_body(q0_ref, q1_ref, b_ref, wl_ref, bl_ref, out_ref):
    o = jnp.maximum(q0_ref[...] + q1_ref[...] + b_ref[...], 0.0)
    out_ref[...] = jnp.dot(o, wl_ref[...],
                           preferred_element_type=jnp.float32) + bl_ref[...]


def _tc_out(q0, q1, b, wl, bl):
    return pl.pallas_call(
        _tc_out_body,
        grid=(N // BN,),
        in_specs=[pl.BlockSpec((BN, C), lambda i: (i, 0)),
                  pl.BlockSpec((BN, C), lambda i: (i, 0)),
                  pl.BlockSpec((1, C), lambda i: (0, 0)),
                  pl.BlockSpec((C, 128), lambda i: (0, 0)),
                  pl.BlockSpec((1, 128), lambda i: (0, 0))],
        out_specs=pl.BlockSpec((BN, 128), lambda i: (i, 0)),
        out_shape=jax.ShapeDtypeStruct((N, 128), jnp.float32),
    )(q0, q1, b, wl, bl)


def _splat(v, lane):
    idx = jnp.full((16,), lane, dtype=jnp.int32)
    return jnp.take(v, idx, mode=lax.GatherScatterMode.PROMISE_IN_BOUNDS)


def _sc_layer_body(emit_ea, *refs):
    if emit_ea:
        (src2, dst2, h_hbm, sdt_hbm, we_hbm, ae_hbm, ewp_hbm,
         part_hbm, ea_hbm,
         sv, dv, srcb, dstb, exb, rows0, rows1, wev, aev, ewpv, zb,
         den_sh, ex_sh, out_sh, gsem, eab) = refs
    else:
        (src2, dst2, h_hbm, sdt_hbm, we_hbm, ae_hbm, ewp_hbm,
         part_hbm,
         sv, dv, srcb, dstb, exb, rows0, rows1, wev, aev, ewpv, zb,
         den_sh, ex_sh, out_sh, gsem) = refs
        ea_hbm = eab = None

    c = lax.axis_index("c")
    tid = lax.axis_index("s")

    def _zb_zero(i, _):
        zb[pl.ds(i * 16, 16)] = jnp.zeros((16,), jnp.float32)
        return 0
    lax.fori_loop(0, 40, _zb_zero, 0)
    pltpu.sync_copy(zb, den_sh.at[pl.ds(tid * 640, 640)])

    def _r0_zero(i, _):
        for j in range(C // 16):
            rows0[i, pl.ds(j * 16, 16)] = jnp.zeros((16,), jnp.float32)
        return 0
    lax.fori_loop(0, RCH, _r0_zero, 0)

    def _out_zero(k, _):
        pltpu.sync_copy(rows0, out_sh.at[pl.ds(tid * 640 + k * RCH, RCH)])
        return 0
    lax.fori_loop(0, 640 // RCH, _out_zero, 0)

    pltpu.sync_copy(we_hbm, wev)
    pltpu.sync_copy(ae_hbm, aev)
    pltpu.sync_copy(ewp_hbm, ewpv)
    pltpu.sync_copy(sdt_hbm.at[0], sv.at[pl.ds(0, N)])
    pltpu.sync_copy(sdt_hbm.at[1], dv.at[pl.ds(0, N)])
    pltpu.sync_copy(src2.at[pl.ds(tid * RA, RA)], srcb)
    pltpu.sync_copy(dst2.at[pl.ds(tid * RA, RA)], dstb)

    acc = wev[pl.ds(0, 16)] * aev[pl.ds(0, 16)]
    for j in range(1, C // 16):
        acc = acc + wev[pl.ds(j * 16, 16)] * aev[pl.ds(j * 16, 16)]
    ce = jnp.sum(acc)
    w0 = ewpv[0]
    w1 = ewpv[1]
    w2 = ewpv[2]
    bew = ewpv[3]

    lanes_f = lax.convert_element_type(
        lax.broadcasted_iota(jnp.int32, (16,), 0), jnp.float32)
    ebase_f = lax.convert_element_type(tid * EPT_A, jnp.float32)

    def _cha(r, _):
        r80f = lax.convert_element_type(r * RCH, jnp.float32)
        for g in range(RCH // 16):
            srcv = srcb[r, pl.ds(g * 16, 16)]
            dstv = dstb[r, pl.ds(g * 16, 16)]
            sg = plsc.load_gather(sv, [srcv])
            dg = plsc.load_gather(dv, [dstv])
            eidf = ebase_f + r80f + (g * 16.0) + lanes_f
            lin = (eidf * w0
                   + lax.convert_element_type(srcv, jnp.float32) * w1
                   + lax.convert_element_type(dstv, jnp.float32) * w2
                   + bew)
            ea = 1.0 / (1.0 + jnp.exp(-lin))
            if emit_ea:
                eab[r, pl.ds(g * 16, 16)] = ea
            alpha = sg + dg + ce * ea
            alpha = jnp.where(alpha >= 0.0, alpha, 0.2 * alpha)
            exb[r, pl.ds(g * 16, 16)] = jnp.exp(alpha)
        pltpu.sync_copy(exb.at[r], den_sh.at[dstb.at[r]], add=True)
        return 0
    lax.fori_loop(0, RA, _cha, 0)

    pltpu.sync_copy(exb, ex_sh.at[pl.ds(tid * RA, RA)])
    if emit_ea:
        @pl.when(c == 0)
        def _():
            pltpu.sync_copy(eab, ea_hbm.at[pl.ds(tid * RA, RA)])

    plsc.subcore_barrier()

    pltpu.sync_copy(den_sh, sv)
    pb_row = c * (NROW // 2) + tid * RB
    pltpu.sync_copy(ex_sh.at[pl.ds(pb_row, RB)], exb.at[pl.ds(0, RB)])
    pltpu.sync_copy(src2.at[pl.ds(pb_row, RB)], srcb.at[pl.ds(0, RB)])
    pltpu.sync_copy(dst2.at[pl.ds(pb_row, RB)], dstb.at[pl.ds(0, RB)])

    def _gstart(k, rbuf):
        return pltpu.async_copy(h_hbm.at[srcb.at[k]], rbuf, gsem)

    def _gwait(k, rbuf):
        pltpu.make_async_copy(h_hbm.at[srcb.at[k]], rbuf, gsem).wait()

    def _process(k, rbuf):
        for g in range(RCH // 16):
            dstv = dstb[k, pl.ds(g * 16, 16)]
            dgv = plsc.load_gather(sv, [dstv])
            exv = exb[k, pl.ds(g * 16, 16)]
            coefv = exv / (dgv + 1e-16)
            for l in range(16):
                rr = g * 16 + l
                cl = _splat(coefv, l)
                for j in range(C // 16):
                    rbuf[rr, pl.ds(j * 16, 16)] = (
                        rbuf[rr, pl.ds(j * 16, 16)] * cl)
        pltpu.sync_copy(rbuf, out_sh.at[dstb.at[k]], add=True)

    _gstart(0, rows0)

    def _pair(i, _):
        k0 = 2 * i
        k1 = 2 * i + 1

        @pl.when(k1 < RB)
        def _():
            _gstart(k1, rows1)

        _gwait(k0, rows0)
        _process(k0, rows0)

        @pl.when(k1 < RB)
        def _():
            @pl.when(k1 + 1 < RB)
            def _():
                _gstart(k1 + 1, rows0)
            _gwait(k1, rows1)
            _process(k1, rows1)
        return 0
    lax.fori_loop(0, (RB + 1) // 2, _pair, 0)

    plsc.subcore_barrier()

    @pl.when(tid < 15)
    def _():
        pltpu.sync_copy(out_sh.at[pl.ds(tid * 640, 640)],
                        part_hbm.at[c, pl.ds(tid * 640, 640)])

    @pl.when(tid == 15)
    def _():
        pltpu.sync_copy(out_sh.at[pl.ds(15 * 640, N - 15 * 640)],
                        part_hbm.at[c, pl.ds(15 * 640, N - 15 * 640)])


def _make_sc_layer(emit_ea):
    out_type = [jax.ShapeDtypeStruct((2, N, C), jnp.float32)]
    if emit_ea:
        out_type.append(jax.ShapeDtypeStruct((NROW, RCH), jnp.float32))
    scratch = [
        pltpu.VMEM((NPAD,), jnp.float32),
        pltpu.VMEM((NPAD,), jnp.float32),
        pltpu.VMEM((RA, RCH), jnp.int32),
        pltpu.VMEM((RA, RCH), jnp.int32),
        pltpu.VMEM((RA, RCH), jnp.float32),
        pltpu.VMEM((RCH, C), jnp.float32),
        pltpu.VMEM((RCH, C), jnp.float32),
        pltpu.VMEM((C,), jnp.float32),
        pltpu.VMEM((C,), jnp.float32),
        pltpu.VMEM((16,), jnp.float32),
        pltpu.VMEM((640,), jnp.float32),
        pltpu.VMEM_SHARED((NPAD,), jnp.float32),
        pltpu.VMEM_SHARED((NROW, RCH), jnp.float32),
        pltpu.VMEM_SHARED((NPAD, C), jnp.float32),
        pltpu.SemaphoreType.DMA,
    ]
    if emit_ea:
        scratch.append(pltpu.VMEM((RA, RCH), jnp.float32))
    return pl.kernel(
        functools.partial(_sc_layer_body, emit_ea),
        out_type=out_type,
        mesh=plsc.VectorSubcoreMesh(**_MESH),
        scratch_types=scratch,
    )


_sc_layer1 = _make_sc_layer(True)
_sc_layer2 = _make_sc_layer(False)


def kernel(x, edge_index, W1, a_s1, a_d1, We1, a_e1, b1,
           W2, a_s2, a_d2, We2, a_e2, b2, Wl, bl, Wew, bew):
    src2 = edge_index[0].reshape(NROW, RCH)
    dst2 = edge_index[1].reshape(NROW, RCH)
    ewp = jnp.concatenate([Wew.reshape(-1), bew.reshape(-1),
                           jnp.zeros((12,), jnp.float32)])
    p1 = jnp.concatenate([a_s1.reshape(1, -1), a_d1.reshape(1, -1),
                          jnp.zeros((6, D), jnp.float32)], axis=0)
    p2 = jnp.concatenate([a_s2.reshape(1, -1), a_d2.reshape(1, -1),
                          jnp.zeros((6, C), jnp.float32)], axis=0)
    wl_pad = jnp.zeros((C, 128), jnp.float32).at[:, 0].set(Wl[:, 0])
    bl_pad = jnp.zeros((1, 128), jnp.float32).at[0, 0].set(bl[0])

    h1, sdt1 = _tc_in(x, W1, p1)
    part1, ea2 = _sc_layer1(src2, dst2, h1, sdt1, We1.reshape(-1),
                            a_e1.reshape(-1), ewp)
    h2, sdt2 = _tc_mid(part1[0], part1[1], b1.reshape(1, C), W2, p2)
    part2 = _sc_layer2(src2, dst2, h2, sdt2, We2.reshape(-1),
                       a_e2.reshape(-1), ewp)
    if isinstance(part2, (list, tuple)):
        part2 = part2[0]
    outp = _tc_out(part2[0], part2[1], b2.reshape(1, C), wl_pad, bl_pad)
    return outp[:, :1], ea2.reshape(E, 1)


# speedup vs baseline: 11.2994x; 11.2994x over previous
import functools

import jax
import jax.numpy as jnp
from jax import lax
from jax.experimental import pallas as pl
from jax.experimental.pallas import tpu as pltpu
from jax.experimental.pallas import tpu_sc as plsc

N = 10000
E = 320000
C = 128
D = 128

NPAD = 10240
ECH = 10000
RB = 125
BLK = 5
NBLK = RB // BLK
BN = 2000

_ROW_B = 80 * C * 4
_SD_B = BLK * 5 * 16 * 4 * 2
_DEN_B = BLK * 5 * 16 * 4
_EA_B = BLK * 80 * 4

_MESH = dict(core_axis_name="c", subcore_axis_name="s", num_cores=2,
             num_subcores=16)


def _tc_in_body(x_ref, w_ref, h_ref):
    h_ref[...] = jnp.dot(x_ref[...], w_ref[...],
                         preferred_element_type=jnp.float32)


def _tc_in(x, w):
    return pl.pallas_call(
        _tc_in_body,
        grid=(N // BN,),
        in_specs=[pl.BlockSpec((BN, D), lambda i: (i, 0)),
                  pl.BlockSpec((D, C), lambda i: (0, 0))],
        out_specs=pl.BlockSpec((BN, C), lambda i: (i, 0)),
        out_shape=jax.ShapeDtypeStruct((N, C), jnp.float32),
    )(x, w)


def _tc_sd_body(p_ref, h_ref, sdt_ref):
    sdt_ref[...] = lax.dot_general(p_ref[...], h_ref[...],
                                   (((1,), (1,)), ((), ())),
                                   preferred_element_type=jnp.float32)


def _tc_sd(p, h):
    return pl.pallas_call(
        _tc_sd_body,
        in_specs=[pl.BlockSpec((8, C), lambda: (0, 0)),
                  pl.BlockSpec((N, C), lambda: (0, 0))],
        out_specs=pl.BlockSpec((8, N), lambda: (0, 0)),
        out_shape=jax.ShapeDtypeStruct((8, N), jnp.float32),
    )(p, h)


def _tc_mid_body(q0_ref, q1_ref, b_ref, w_ref, h_ref):
    o = jnp.maximum(q0_ref[...] + q1_ref[...] + b_ref[...], 0.0)
    h_ref[...] = jnp.dot(o, w_ref[...], preferred_element_type=jnp.float32)


def _tc_mid(q0, q1, b, w):
    return pl.pallas_call(
        _tc_mid_body,
        grid=(N // BN,),
        in_specs=[pl.BlockSpec((BN, C), lambda i: (i, 0)),
                  pl.BlockSpec((BN, C), lambda i: (i, 0)),
                  pl.BlockSpec((1, C), lambda i: (0, 0)),
                  pl.BlockSpec((C, C), lambda i: (0, 0))],
        out_specs=pl.BlockSpec((BN, C), lambda i: (i, 0)),
        out_shape=jax.ShapeDtypeStruct((N, C), jnp.float32),
    )(q0, q1, b, w)


def _tc_out_body(q0_ref, q1_ref, b_ref, wl_ref, bl_ref, out_ref):
    o = jnp.maximum(q0_ref[...] + q1_ref[...] + b_ref[...], 0.0)
    out_ref[...] = jnp.dot(o, wl_ref[...],
                           preferred_element_type=jnp.float32) + bl_ref[...]


def _tc_out(q0, q1, b, wl, bl):
    return pl.pallas_call(
        _tc_out_body,
        grid=(N // BN,),
        in_specs=[pl.BlockSpec((BN, C), lambda i: (i, 0)),
                  pl.BlockSpec((BN, C), lambda i: (i, 0)),
                  pl.BlockSpec((1, C), lambda i: (0, 0)),
                  pl.BlockSpec((C, 128), lambda i: (0, 0)),
                  pl.BlockSpec((1, 128), lambda i: (0, 0))],
        out_specs=pl.BlockSpec((BN, 128), lambda i: (i, 0)),
        out_shape=jax.ShapeDtypeStruct((N, 128), jnp.float32),
    )(q0, q1, b, wl, bl)


def _splat(v, lane):
    idx = jnp.full((16, 1), lane, dtype=jnp.int32)
    dn = lax.GatherDimensionNumbers(offset_dims=(), collapsed_slice_dims=(0,),
                                    start_index_map=(0,))
    return lax.gather(v, idx, dn, slice_sizes=(1,),
                      mode=lax.GatherScatterMode.PROMISE_IN_BOUNDS)


def _sc_layer_body(emit_ea, *refs):
    if emit_ea:
        (src1, dst1, h_hbm, s_hbm, d_hbm, we_hbm, ae_hbm, ewp_hbm,
         part_hbm, ea_hbm,
         srcb, dstb, rows0, rows1, sblk, dblk, exblk, dgblk, dst2a, dst2b,
         wev, aev, ewpv,
         den_sh, out_sh,
         gsem, ssem, bsem, easem, eablk) = refs
    else:
        (src1, dst1, h_hbm, s_hbm, d_hbm, we_hbm, ae_hbm, ewp_hbm,
         part_hbm,
         srcb, dstb, rows0, rows1, sblk, dblk, exblk, dgblk, dst2a, dst2b,
         wev, aev, ewpv,
         den_sh, out_sh,
         gsem, ssem, bsem, easem) = refs
        ea_hbm = eablk = None

    c = lax.axis_index("c")
    tid = lax.axis_index("s")

    def _ex_zero(i, _):
        exblk[pl.ds(i * 16, 16)] = jnp.zeros((16,), jnp.float32)
        return 0
    lax.fori_loop(0, 25, _ex_zero, 0)
    pltpu.sync_copy(exblk, den_sh.at[pl.ds(tid * 640, 400)])
    pltpu.sync_copy(exblk.at[pl.ds(0, 240)],
                    den_sh.at[pl.ds(tid * 640 + 400, 240)])

    def _r0_zero(i, _):
        for j in range(C // 16):
            rows0[i, pl.ds(j * 16, 16)] = jnp.zeros((16,), jnp.float32)
        return 0
    lax.fori_loop(0, 80, _r0_zero, 0)

    def _out_zero(k, _):
        pltpu.sync_copy(rows0, out_sh.at[pl.ds(tid * 640 + k * 80, 80)])
        return 0
    lax.fori_loop(0, 8, _out_zero, 0)

    pltpu.sync_copy(we_hbm, wev)
    pltpu.sync_copy(ae_hbm, aev)
    pltpu.sync_copy(ewp_hbm, ewpv)

    acc = wev[pl.ds(0, 16)] * aev[pl.ds(0, 16)]
    for j in range(1, C // 16):
        acc = acc + wev[pl.ds(j * 16, 16)] * aev[pl.ds(j * 16, 16)]
    ce = _splat(acc, 0)
    for l in range(1, 16):
        ce = ce + _splat(acc, l)

    ewv = ewpv[pl.ds(0, 16)]
    w0 = ewv[0]
    w1 = ewv[1]
    w2 = ewv[2]
    bew = ewv[3]

    lanes_f = lax.convert_element_type(
        lax.broadcasted_iota(jnp.int32, (16,), 0), jnp.float32)

    plsc.subcore_barrier()

    def _edge_group(ebase_f, off, sg, dg):
        srcv = srcb[pl.ds(off, 16)]
        dstv = dstb[pl.ds(off, 16)]
        eidf = ebase_f + lax.convert_element_type(off, jnp.float32) + lanes_f
        lin = (eidf * w0
               + lax.convert_element_type(srcv, jnp.float32) * w1
               + lax.convert_element_type(dstv, jnp.float32) * w2
               + bew)
        ea = 1.0 / (1.0 + jnp.exp(-lin))
        alpha = sg + dg + ce * ea
        alpha = jnp.where(alpha >= 0.0, alpha, 0.2 * alpha)
        ex = jnp.exp(alpha)
        return dstv, ea, ex

    for half in range(2):
        chunk = 2 * tid + half
        base = chunk * ECH
        pltpu.sync_copy(src1.at[pl.ds(base, ECH)], srcb)
        pltpu.sync_copy(dst1.at[pl.ds(base, ECH)], dstb)
        ebase_f = lax.convert_element_type(base, jnp.float32)

        def _blk_compute(b, slot, half=half, ebase_f=ebase_f):
            for r in range(BLK):
                o = b * BLK * 80 + r * 80
                so = slot * 400 + r * 80
                pltpu.async_copy(s_hbm.at[srcb.at[pl.ds(o, 80)]],
                                 sblk.at[pl.ds(so, 80)], ssem).wait()
                pltpu.async_copy(d_hbm.at[dstb.at[pl.ds(o, 80)]],
                                 dblk.at[pl.ds(so, 80)], ssem).wait()
            for r in range(BLK):
                o = b * BLK * 80 + r * 80
                for g in range(5):
                    off = o + g * 16
                    so = slot * 400 + r * 80 + g * 16
                    sg = sblk[pl.ds(so, 16)]
                    dg = dblk[pl.ds(so, 16)]
                    dstv, ea, ex = _edge_group(ebase_f, off, sg, dg)
                    dst2a[r, pl.ds(g * 16, 16)] = dstv
                    if emit_ea:
                        eablk[pl.ds(so, 16)] = ea
                    exblk[pl.ds(r * 80 + g * 16, 16)] = ex
                pltpu.sync_copy(exblk.at[pl.ds(r * 80, 80)],
                                den_sh.at[dst2a.at[r]], add=True)
            if emit_ea:
                @pl.when(c == 0)
                def _():
                    pltpu.async_copy(
                        eablk.at[pl.ds(slot * 400, 400)],
                        ea_hbm.at[pl.ds(base + b * 400, 400)], easem).wait()

        def _ablk(b, _):
            _blk_compute(b, 0)
            return 0
        lax.fori_loop(0, NBLK, _ablk, 0)

    plsc.subcore_barrier()

    wid = c * 16 + tid
    base_b = wid * ECH
    pltpu.sync_copy(src1.at[pl.ds(base_b, ECH)], srcb)
    pltpu.sync_copy(dst1.at[pl.ds(base_b, ECH)], dstb)
    ebase_b = lax.convert_element_type(base_b, jnp.float32)

    def _b_process(k, rbuf, slot):
        so = slot * 80
        pltpu.async_copy(h_hbm.at[srcb.at[pl.ds(k * 80, 80)]],
                         rbuf, gsem).wait()
        pltpu.async_copy(s_hbm.at[srcb.at[pl.ds(k * 80, 80)]],
                         sblk.at[pl.ds(so, 80)], bsem).wait()
        pltpu.async_copy(d_hbm.at[dstb.at[pl.ds(k * 80, 80)]],
                         dblk.at[pl.ds(so, 80)], bsem).wait()
        pltpu.async_copy(den_sh.at[dstb.at[pl.ds(k * 80, 80)]],
                         dgblk.at[pl.ds(so, 80)], bsem).wait()
        for g in range(5):
            so = slot * 80 + g * 16
            sg = sblk[pl.ds(so, 16)]
            dg = dblk[pl.ds(so, 16)]
            dstv, _, ex = _edge_group(ebase_b, k * 80 + g * 16, sg, dg)
            dst2b[slot, pl.ds(g * 16, 16)] = dstv
            den = dgblk[pl.ds(so, 16)]
            coefv = ex / (den + 1e-16)
            for l in range(16):
                rr = g * 16 + l
                cl = _splat(coefv, l)
                for j in range(C // 16):
                    rbuf[rr, pl.ds(j * 16, 16)] = (
                        rbuf[rr, pl.ds(j * 16, 16)] * cl)
        pltpu.sync_copy(rbuf, out_sh.at[dst2b.at[slot]], add=True)

    def _bchunk(k, _):
        _b_process(k, rows0, 0)
        return 0
    lax.fori_loop(0, RB, _bchunk, 0)

    plsc.subcore_barrier()

    @pl.when(tid < 15)
    def _():
        pltpu.sync_copy(out_sh.at[pl.ds(tid * 640, 640)],
                        part_hbm.at[c, pl.ds(tid * 640, 640)])

    @pl.when(tid == 15)
    def _():
        pltpu.sync_copy(out_sh.at[pl.ds(15 * 640, N - 15 * 640)],
                        part_hbm.at[c, pl.ds(15 * 640, N - 15 * 640)])


def _make_sc_layer(emit_ea):
    out_type = [jax.ShapeDtypeStruct((2, N, C), jnp.float32)]
    if emit_ea:
        out_type.append(jax.ShapeDtypeStruct((E,), jnp.float32))
    scratch = [
        pltpu.VMEM((ECH,), jnp.int32),
        pltpu.VMEM((ECH,), jnp.int32),
        pltpu.VMEM((80, C), jnp.float32),
        pltpu.VMEM((80, C), jnp.float32),
        pltpu.VMEM((800,), jnp.float32),
        pltpu.VMEM((800,), jnp.float32),
        pltpu.VMEM((400,), jnp.float32),
        pltpu.VMEM((160,), jnp.float32),
        pltpu.VMEM((BLK, 80), jnp.int32),
        pltpu.VMEM((2, 80), jnp.int32),
        pltpu.VMEM((C,), jnp.float32),
        pltpu.VMEM((C,), jnp.float32),
        pltpu.VMEM((16,), jnp.float32),
        pltpu.VMEM_SHARED((NPAD,), jnp.float32),
        pltpu.VMEM_SHARED((NPAD, C), jnp.float32),
        pltpu.SemaphoreType.DMA,
        pltpu.SemaphoreType.DMA,
        pltpu.SemaphoreType.DMA,
        pltpu.SemaphoreType.DMA,
    ]
    if emit_ea:
        scratch.append(pltpu.VMEM((800,), jnp.float32))
    return pl.kernel(
        functools.partial(_sc_layer_body, emit_ea),
        out_type=out_type,
        mesh=plsc.VectorSubcoreMesh(**_MESH),
        scratch_types=scratch,
        compiler_params=pltpu.CompilerParams(needs_layout_passes=False),
    )


_sc_layer1 = _make_sc_layer(True)
_sc_layer2 = _make_sc_layer(False)


def kernel(x, edge_index, W1, a_s1, a_d1, We1, a_e1, b1,
           W2, a_s2, a_d2, We2, a_e2, b2, Wl, bl, Wew, bew):
    src1 = edge_index[0]
    dst1 = edge_index[1]
    ewp = jnp.concatenate([Wew.reshape(-1), bew.reshape(-1),
                           jnp.zeros((12,), jnp.float32)])
    p1 = jnp.concatenate([a_s1.reshape(1, -1), a_d1.reshape(1, -1),
                          jnp.zeros((6, D), jnp.float32)], axis=0)
    p2 = jnp.concatenate([a_s2.reshape(1, -1), a_d2.reshape(1, -1),
                          jnp.zeros((6, C), jnp.float32)], axis=0)
    wl_pad = jnp.zeros((C, 128), jnp.float32).at[:, 0].set(Wl[:, 0])
    bl_pad = jnp.zeros((1, 128), jnp.float32).at[0, 0].set(bl[0])

    h1 = _tc_in(x, W1)
    sdt1 = _tc_sd(p1, h1)
    part1, ea1 = _sc_layer1(src1, dst1, h1, sdt1[0], sdt1[1],
                            We1.reshape(-1), a_e1.reshape(-1), ewp)
    h2 = _tc_mid(part1[0], part1[1], b1.reshape(1, C), W2)
    sdt2 = _tc_sd(p2, h2)
    part2 = _sc_layer2(src1, dst1, h2, sdt2[0], sdt2[1],
                       We2.reshape(-1), a_e2.reshape(-1), ewp)
    if isinstance(part2, (list, tuple)):
        part2 = part2[0]
    outp = _tc_out(part2[0], part2[1], b2.reshape(1, C), wl_pad, bl_pad)
    return outp[:, :1], ea1.reshape(E, 1)


# speedup vs baseline: 14.4447x; 1.2784x over previous
import functools

import jax
import jax.numpy as jnp
from jax import lax
from jax.experimental import pallas as pl
from jax.experimental.pallas import tpu as pltpu
from jax.experimental.pallas import tpu_sc as plsc

N = 10000
E = 320000
C = 128
D = 128

NPAD = 10240
ECH = 10000
RB = 125
BLK = 5
NBLK = RB // BLK
BN = 2000

_ROW_B = 80 * C * 4
_SD_B = BLK * 5 * 16 * 4 * 2
_DEN_B = BLK * 5 * 16 * 4
_EA_B = BLK * 80 * 4

_MESH = dict(core_axis_name="c", subcore_axis_name="s", num_cores=2,
             num_subcores=16)


def _tc_in_body(x_ref, w_ref, h_ref):
    h_ref[...] = jnp.dot(x_ref[...], w_ref[...],
                         preferred_element_type=jnp.float32)


def _tc_in(x, w):
    return pl.pallas_call(
        _tc_in_body,
        grid=(N // BN,),
        in_specs=[pl.BlockSpec((BN, D), lambda i: (i, 0)),
                  pl.BlockSpec((D, C), lambda i: (0, 0))],
        out_specs=pl.BlockSpec((BN, C), lambda i: (i, 0)),
        out_shape=jax.ShapeDtypeStruct((N, C), jnp.float32),
    )(x, w)


def _tc_sd_body(p_ref, h_ref, sdt_ref):
    sdt_ref[...] = lax.dot_general(p_ref[...], h_ref[...],
                                   (((1,), (1,)), ((), ())),
                                   preferred_element_type=jnp.float32)


def _tc_sd(p, h):
    return pl.pallas_call(
        _tc_sd_body,
        in_specs=[pl.BlockSpec((8, C), lambda: (0, 0)),
                  pl.BlockSpec((N, C), lambda: (0, 0))],
        out_specs=pl.BlockSpec((8, N), lambda: (0, 0)),
        out_shape=jax.ShapeDtypeStruct((8, N), jnp.float32),
    )(p, h)


def _tc_mid_body(q0_ref, q1_ref, b_ref, w_ref, h_ref):
    o = jnp.maximum(q0_ref[...] + q1_ref[...] + b_ref[...], 0.0)
    h_ref[...] = jnp.dot(o, w_ref[...], preferred_element_type=jnp.float32)


def _tc_mid(q0, q1, b, w):
    return pl.pallas_call(
        _tc_mid_body,
        grid=(N // BN,),
        in_specs=[pl.BlockSpec((BN, C), lambda i: (i, 0)),
                  pl.BlockSpec((BN, C), lambda i: (i, 0)),
                  pl.BlockSpec((1, C), lambda i: (0, 0)),
                  pl.BlockSpec((C, C), lambda i: (0, 0))],
        out_specs=pl.BlockSpec((BN, C), lambda i: (i, 0)),
        out_shape=jax.ShapeDtypeStruct((N, C), jnp.float32),
    )(q0, q1, b, w)


def _tc_out_body(q0_ref, q1_ref, b_ref, wl_ref, bl_ref, out_ref):
    o = jnp.maximum(q0_ref[...] + q1_ref[...] + b_ref[...], 0.0)
    out_ref[...] = jnp.dot(o, wl_ref[...],
                           preferred_element_type=jnp.float32) + bl_ref[...]


def _tc_out(q0, q1, b, wl, bl):
    return pl.pallas_call(
        _tc_out_body,
        grid=(N // BN,),
        in_specs=[pl.BlockSpec((BN, C), lambda i: (i, 0)),
                  pl.BlockSpec((BN, C), lambda i: (i, 0)),
                  pl.BlockSpec((1, C), lambda i: (0, 0)),
                  pl.BlockSpec((C, 128), lambda i: (0, 0)),
                  pl.BlockSpec((1, 128), lambda i: (0, 0))],
        out_specs=pl.BlockSpec((BN, 128), lambda i: (i, 0)),
        out_shape=jax.ShapeDtypeStruct((N, 128), jnp.float32),
    )(q0, q1, b, wl, bl)


def _splat(v, lane):
    idx = jnp.full((16, 1), lane, dtype=jnp.int32)
    dn = lax.GatherDimensionNumbers(offset_dims=(), collapsed_slice_dims=(0,),
                                    start_index_map=(0,))
    return lax.gather(v, idx, dn, slice_sizes=(1,),
                      mode=lax.GatherScatterMode.PROMISE_IN_BOUNDS)


def _sc_layer_body(emit_ea, *refs):
    if emit_ea:
        (src1, dst1, h_hbm, s_hbm, d_hbm, we_hbm, ae_hbm, ewp_hbm,
         part_hbm, ex_hbm, ea_hbm,
         srcb, dstb, rows0, rows1, sblk, dblk, exblk, dgblk, dst2a, dst2b,
         exrd, wev, aev, ewpv,
         den_sh, out_sh,
         gsem, ssem, bsem, easem, eablk) = refs
    else:
        (src1, dst1, h_hbm, s_hbm, d_hbm, we_hbm, ae_hbm, ewp_hbm,
         part_hbm, ex_hbm,
         srcb, dstb, rows0, rows1, sblk, dblk, exblk, dgblk, dst2a, dst2b,
         exrd, wev, aev, ewpv,
         den_sh, out_sh,
         gsem, ssem, bsem, easem) = refs
        ea_hbm = eablk = None

    c = lax.axis_index("c")
    tid = lax.axis_index("s")

    def _ex_zero(i, _):
        exblk[pl.ds(i * 16, 16)] = jnp.zeros((16,), jnp.float32)
        return 0
    lax.fori_loop(0, 25, _ex_zero, 0)
    pltpu.sync_copy(exblk.at[pl.ds(0, 400)], den_sh.at[pl.ds(tid * 640, 400)])
    pltpu.sync_copy(exblk.at[pl.ds(0, 240)],
                    den_sh.at[pl.ds(tid * 640 + 400, 240)])

    def _r0_zero(i, _):
        for j in range(C // 16):
            rows0[i, pl.ds(j * 16, 16)] = jnp.zeros((16,), jnp.float32)
        return 0
    lax.fori_loop(0, 80, _r0_zero, 0)

    def _out_zero(k, _):
        pltpu.sync_copy(rows0, out_sh.at[pl.ds(tid * 640 + k * 80, 80)])
        return 0
    lax.fori_loop(0, 8, _out_zero, 0)

    pltpu.sync_copy(we_hbm, wev)
    pltpu.sync_copy(ae_hbm, aev)
    pltpu.sync_copy(ewp_hbm, ewpv)

    acc = wev[pl.ds(0, 16)] * aev[pl.ds(0, 16)]
    for j in range(1, C // 16):
        acc = acc + wev[pl.ds(j * 16, 16)] * aev[pl.ds(j * 16, 16)]
    ce = _splat(acc, 0)
    for l in range(1, 16):
        ce = ce + _splat(acc, l)

    ewv = ewpv[pl.ds(0, 16)]
    w0 = ewv[0]
    w1 = ewv[1]
    w2 = ewv[2]
    bew = ewv[3]

    lanes_f = lax.convert_element_type(
        lax.broadcasted_iota(jnp.int32, (16,), 0), jnp.float32)

    plsc.subcore_barrier()

    def _edge_group(ebase_f, off, sg, dg):
        srcv = srcb[pl.ds(off, 16)]
        dstv = dstb[pl.ds(off, 16)]
        eidf = ebase_f + lax.convert_element_type(off, jnp.float32) + lanes_f
        lin = (eidf * w0
               + lax.convert_element_type(srcv, jnp.float32) * w1
               + lax.convert_element_type(dstv, jnp.float32) * w2
               + bew)
        ea = 1.0 / (1.0 + jnp.exp(-lin))
        alpha = sg + dg + ce * ea
        alpha = jnp.where(alpha >= 0.0, alpha, 0.2 * alpha)
        ex = jnp.exp(alpha)
        return dstv, ea, ex

    for half in range(2):
        chunk = 2 * tid + half
        base = chunk * ECH
        pltpu.sync_copy(src1.at[pl.ds(base, ECH)], srcb)
        pltpu.sync_copy(dst1.at[pl.ds(base, ECH)], dstb)
        ebase_f = lax.convert_element_type(base, jnp.float32)
        own = (chunk >= 16) == (c == 1)

        def _blk_compute(b, slot, own=own, base=base, ebase_f=ebase_f):
            for r in range(BLK):
                o = b * 400 + r * 80
                so = slot * 400 + r * 80
                pltpu.async_copy(s_hbm.at[srcb.at[pl.ds(o, 80)]],
                                 sblk.at[pl.ds(so, 80)], ssem).wait()
                pltpu.async_copy(d_hbm.at[dstb.at[pl.ds(o, 80)]],
                                 dblk.at[pl.ds(so, 80)], ssem).wait()
            for r in range(BLK):
                o = b * 400 + r * 80
                for g in range(5):
                    so = slot * 400 + r * 80 + g * 16
                    sg = sblk[pl.ds(so, 16)]
                    dg = dblk[pl.ds(so, 16)]
                    dstv, ea, ex = _edge_group(ebase_f, o + g * 16, sg, dg)
                    dst2a[slot * BLK + r, pl.ds(g * 16, 16)] = dstv
                    if emit_ea:
                        eablk[pl.ds(so, 16)] = ea
                    exblk[pl.ds(so, 16)] = ex
                pltpu.sync_copy(exblk.at[pl.ds(slot * 400 + r * 80, 80)],
                                den_sh.at[dst2a.at[slot * BLK + r]],
                                add=True)
            @pl.when(own)
            def _():
                pltpu.async_copy(exblk.at[pl.ds(slot * 400, 400)],
                                 ex_hbm.at[pl.ds(base + b * 400, 400)],
                                 easem).wait()
            if emit_ea:
                @pl.when(c == 0)
                def _():
                    pltpu.async_copy(
                        eablk.at[pl.ds(slot * 400, 400)],
                        ea_hbm.at[pl.ds(base + b * 400, 400)], easem).wait()

        def _ablk(b, _):
            _blk_compute(b, 0)
            return 0
        lax.fori_loop(0, NBLK, _ablk, 0)

    plsc.subcore_barrier()

    wid = c * 16 + tid
    base_b = wid * ECH
    pltpu.sync_copy(src1.at[pl.ds(base_b, ECH)], srcb)
    pltpu.sync_copy(dst1.at[pl.ds(base_b, ECH)], dstb)

    def _b_process(k, rbuf, slot, gdesc):
        so = slot * 80
        pltpu.async_copy(ex_hbm.at[pl.ds(base_b + k * 80, 80)],
                         exrd.at[pl.ds(so, 80)], bsem).wait()
        pltpu.async_copy(den_sh.at[dstb.at[pl.ds(k * 80, 80)]],
                         dgblk.at[pl.ds(so, 80)], bsem).wait()
        gdesc.wait()
        for g in range(5):
            so = slot * 80 + g * 16
            dstv = dstb[pl.ds(k * 80 + g * 16, 16)]
            dst2b[slot, pl.ds(g * 16, 16)] = dstv
            ex = exrd[pl.ds(so, 16)]
            den = dgblk[pl.ds(so, 16)]
            coefv = ex / (den + 1e-16)
            for l in range(16):
                rr = g * 16 + l
                cl = _splat(coefv, l)
                for j in range(C // 16):
                    rbuf[rr, pl.ds(j * 16, 16)] = (
                        rbuf[rr, pl.ds(j * 16, 16)] * cl)
        pltpu.sync_copy(rbuf, out_sh.at[dst2b.at[slot]], add=True)

    def _grow(k, rbuf):
        return pltpu.async_copy(h_hbm.at[srcb.at[pl.ds(k * 80, 80)]],
                                rbuf, gsem)

    def _bpair(i, _):
        k0 = 2 * i
        k1 = 2 * i + 1
        d0 = _grow(k0, rows0)
        d1 = _grow(k1, rows1)
        _b_process(k0, rows0, 0, d0)
        _b_process(k1, rows1, 1, d1)
        return 0
    lax.fori_loop(0, (RB - 1) // 2, _bpair, 0)
    dt = _grow(RB - 1, rows0)
    _b_process(RB - 1, rows0, 0, dt)

    plsc.subcore_barrier()

    @pl.when(tid < 15)
    def _():
        pltpu.sync_copy(out_sh.at[pl.ds(tid * 640, 640)],
                        part_hbm.at[c, pl.ds(tid * 640, 640)])

    @pl.when(tid == 15)
    def _():
        pltpu.sync_copy(out_sh.at[pl.ds(15 * 640, N - 15 * 640)],
                        part_hbm.at[c, pl.ds(15 * 640, N - 15 * 640)])


def _make_sc_layer(emit_ea):
    out_type = [jax.ShapeDtypeStruct((2, N, C), jnp.float32),
                jax.ShapeDtypeStruct((E,), jnp.float32)]
    if emit_ea:
        out_type.append(jax.ShapeDtypeStruct((E,), jnp.float32))
    scratch = [
        pltpu.VMEM((ECH,), jnp.int32),
        pltpu.VMEM((ECH,), jnp.int32),
        pltpu.VMEM((80, C), jnp.float32),
        pltpu.VMEM((80, C), jnp.float32),
        pltpu.VMEM((800,), jnp.float32),
        pltpu.VMEM((800,), jnp.float32),
        pltpu.VMEM((800,), jnp.float32),
        pltpu.VMEM((160,), jnp.float32),
        pltpu.VMEM((2 * BLK, 80), jnp.int32),
        pltpu.VMEM((2, 80), jnp.int32),
        pltpu.VMEM((160,), jnp.float32),
        pltpu.VMEM((C,), jnp.float32),
        pltpu.VMEM((C,), jnp.float32),
        pltpu.VMEM((16,), jnp.float32),
        pltpu.VMEM_SHARED((NPAD,), jnp.float32),
        pltpu.VMEM_SHARED((NPAD, C), jnp.float32),
        pltpu.SemaphoreType.DMA,
        pltpu.SemaphoreType.DMA,
        pltpu.SemaphoreType.DMA,
        pltpu.SemaphoreType.DMA,
    ]
    if emit_ea:
        scratch.append(pltpu.VMEM((800,), jnp.float32))
    return pl.kernel(
        functools.partial(_sc_layer_body, emit_ea),
        out_type=out_type,
        mesh=plsc.VectorSubcoreMesh(**_MESH),
        scratch_types=scratch,
        compiler_params=pltpu.CompilerParams(needs_layout_passes=False),
    )


_sc_layer1 = _make_sc_layer(True)
_sc_layer2 = _make_sc_layer(False)


def kernel(x, edge_index, W1, a_s1, a_d1, We1, a_e1, b1,
           W2, a_s2, a_d2, We2, a_e2, b2, Wl, bl, Wew, bew):
    src1 = edge_index[0]
    dst1 = edge_index[1]
    ewp = jnp.concatenate([Wew.reshape(-1), bew.reshape(-1),
                           jnp.zeros((12,), jnp.float32)])
    p1 = jnp.concatenate([a_s1.reshape(1, -1), a_d1.reshape(1, -1),
                          jnp.zeros((6, D), jnp.float32)], axis=0)
    p2 = jnp.concatenate([a_s2.reshape(1, -1), a_d2.reshape(1, -1),
                          jnp.zeros((6, C), jnp.float32)], axis=0)
    wl_pad = jnp.zeros((C, 128), jnp.float32).at[:, 0].set(Wl[:, 0])
    bl_pad = jnp.zeros((1, 128), jnp.float32).at[0, 0].set(bl[0])

    h1 = _tc_in(x, W1)
    sdt1 = _tc_sd(p1, h1)
    part1, _, ea1 = _sc_layer1(src1, dst1, h1, sdt1[0], sdt1[1],
                               We1.reshape(-1), a_e1.reshape(-1), ewp)
    h2 = _tc_mid(part1[0], part1[1], b1.reshape(1, C), W2)
    sdt2 = _tc_sd(p2, h2)
    part2, _ = _sc_layer2(src1, dst1, h2, sdt2[0], sdt2[1],
                          We2.reshape(-1), a_e2.reshape(-1), ewp)
    outp = _tc_out(part2[0], part2[1], b2.reshape(1, C), wl_pad, bl_pad)
    return outp[:, :1], ea1.reshape(E, 1)


# speedup vs baseline: 14.4941x; 1.0034x over previous
import functools

import jax
import jax.numpy as jnp
from jax import lax
from jax.experimental import pallas as pl
from jax.experimental.pallas import tpu as pltpu
from jax.experimental.pallas import tpu_sc as plsc

N = 10000
E = 320000
C = 128
D = 128

NPAD = 10240
ECH = 10000
RB = 125
BLK = 5
NBLK = RB // BLK
BN = 2000

_ROW_B = 80 * C * 4
_SD_B = BLK * 5 * 16 * 4 * 2
_DEN_B = BLK * 5 * 16 * 4
_EA_B = BLK * 80 * 4

_MESH = dict(core_axis_name="c", subcore_axis_name="s", num_cores=2,
             num_subcores=16)


def _tc_in_body(x_ref, w_ref, h_ref):
    h_ref[...] = jnp.dot(x_ref[...], w_ref[...],
                         preferred_element_type=jnp.float32)


def _tc_in(x, w):
    return pl.pallas_call(
        _tc_in_body,
        grid=(N // BN,),
        in_specs=[pl.BlockSpec((BN, D), lambda i: (i, 0)),
                  pl.BlockSpec((D, C), lambda i: (0, 0))],
        out_specs=pl.BlockSpec((BN, C), lambda i: (i, 0)),
        out_shape=jax.ShapeDtypeStruct((N, C), jnp.float32),
    )(x, w)


def _tc_sd_body(p_ref, h_ref, sdt_ref):
    sdt_ref[...] = lax.dot_general(p_ref[...], h_ref[...],
                                   (((1,), (1,)), ((), ())),
                                   preferred_element_type=jnp.float32)


def _tc_sd(p, h):
    return pl.pallas_call(
        _tc_sd_body,
        in_specs=[pl.BlockSpec((8, C), lambda: (0, 0)),
                  pl.BlockSpec((N, C), lambda: (0, 0))],
        out_specs=pl.BlockSpec((8, N), lambda: (0, 0)),
        out_shape=jax.ShapeDtypeStruct((8, N), jnp.float32),
    )(p, h)


def _tc_mid_body(q0_ref, q1_ref, b_ref, w_ref, h_ref):
    o = jnp.maximum(q0_ref[...] + q1_ref[...] + b_ref[...], 0.0)
    h_ref[...] = jnp.dot(o, w_ref[...], preferred_element_type=jnp.float32)


def _tc_mid(q0, q1, b, w):
    return pl.pallas_call(
        _tc_mid_body,
        grid=(N // BN,),
        in_specs=[pl.BlockSpec((BN, C), lambda i: (i, 0)),
                  pl.BlockSpec((BN, C), lambda i: (i, 0)),
                  pl.BlockSpec((1, C), lambda i: (0, 0)),
                  pl.BlockSpec((C, C), lambda i: (0, 0))],
        out_specs=pl.BlockSpec((BN, C), lambda i: (i, 0)),
        out_shape=jax.ShapeDtypeStruct((N, C), jnp.float32),
    )(q0, q1, b, w)


def _tc_out_body(q0_ref, q1_ref, b_ref, wl_ref, bl_ref, out_ref):
    o = jnp.maximum(q0_ref[...] + q1_ref[...] + b_ref[...], 0.0)
    out_ref[...] = jnp.dot(o, wl_ref[...],
                           preferred_element_type=jnp.float32) + bl_ref[...]


def _tc_out(q0, q1, b, wl, bl):
    return pl.pallas_call(
        _tc_out_body,
        grid=(N // BN,),
        in_specs=[pl.BlockSpec((BN, C), lambda i: (i, 0)),
                  pl.BlockSpec((BN, C), lambda i: (i, 0)),
                  pl.BlockSpec((1, C), lambda i: (0, 0)),
                  pl.BlockSpec((C, 128), lambda i: (0, 0)),
                  pl.BlockSpec((1, 128), lambda i: (0, 0))],
        out_specs=pl.BlockSpec((BN, 128), lambda i: (i, 0)),
        out_shape=jax.ShapeDtypeStruct((N, 128), jnp.float32),
    )(q0, q1, b, wl, bl)


def _splat(v, lane):
    idx = jnp.full((16, 1), lane, dtype=jnp.int32)
    dn = lax.GatherDimensionNumbers(offset_dims=(), collapsed_slice_dims=(0,),
                                    start_index_map=(0,))
    return lax.gather(v, idx, dn, slice_sizes=(1,),
                      mode=lax.GatherScatterMode.PROMISE_IN_BOUNDS)


def _sc_layer_body(emit_ea, *refs):
    if emit_ea:
        (src1, dst1, h_hbm, s_hbm, d_hbm, we_hbm, ae_hbm, ewp_hbm,
         part_hbm, ex_hbm, ea_hbm,
         srcb, dstb, rows0, rows1, sblk, dblk, exblk, dgblk, dst2a, dst2b,
         exrd, wev, aev, ewpv,
         den_sh, out_sh,
         gsem, ssem, bsem, easem, eablk) = refs
    else:
        (src1, dst1, h_hbm, s_hbm, d_hbm, we_hbm, ae_hbm, ewp_hbm,
         part_hbm, ex_hbm,
         srcb, dstb, rows0, rows1, sblk, dblk, exblk, dgblk, dst2a, dst2b,
         exrd, wev, aev, ewpv,
         den_sh, out_sh,
         gsem, ssem, bsem, easem) = refs
        ea_hbm = eablk = None

    c = lax.axis_index("c")
    tid = lax.axis_index("s")

    def _ex_zero(i, _):
        exblk[pl.ds(i * 16, 16)] = jnp.zeros((16,), jnp.float32)
        return 0
    lax.fori_loop(0, 25, _ex_zero, 0)
    pltpu.sync_copy(exblk.at[pl.ds(0, 400)], den_sh.at[pl.ds(tid * 640, 400)])
    pltpu.sync_copy(exblk.at[pl.ds(0, 240)],
                    den_sh.at[pl.ds(tid * 640 + 400, 240)])

    def _r0_zero(i, _):
        for j in range(C // 16):
            rows0[i, pl.ds(j * 16, 16)] = jnp.zeros((16,), jnp.float32)
        return 0
    lax.fori_loop(0, 80, _r0_zero, 0)

    def _out_zero(k, _):
        pltpu.sync_copy(rows0, out_sh.at[pl.ds(tid * 640 + k * 80, 80)])
        return 0
    lax.fori_loop(0, 8, _out_zero, 0)

    pltpu.sync_copy(we_hbm, wev)
    pltpu.sync_copy(ae_hbm, aev)
    pltpu.sync_copy(ewp_hbm, ewpv)

    acc = wev[pl.ds(0, 16)] * aev[pl.ds(0, 16)]
    for j in range(1, C // 16):
        acc = acc + wev[pl.ds(j * 16, 16)] * aev[pl.ds(j * 16, 16)]
    ce = _splat(acc, 0)
    for l in range(1, 16):
        ce = ce + _splat(acc, l)

    ewv = ewpv[pl.ds(0, 16)]
    w0 = ewv[0]
    w1 = ewv[1]
    w2 = ewv[2]
    bew = ewv[3]

    lanes_f = lax.convert_element_type(
        lax.broadcasted_iota(jnp.int32, (16,), 0), jnp.float32)

    plsc.subcore_barrier()

    def _edge_group(ebase_f, off, sg, dg):
        srcv = srcb[pl.ds(off, 16)]
        dstv = dstb[pl.ds(off, 16)]
        eidf = ebase_f + lax.convert_element_type(off, jnp.float32) + lanes_f
        lin = (eidf * w0
               + lax.convert_element_type(srcv, jnp.float32) * w1
               + lax.convert_element_type(dstv, jnp.float32) * w2
               + bew)
        ea = 1.0 / (1.0 + jnp.exp(-lin))
        alpha = sg + dg + ce * ea
        alpha = jnp.where(alpha >= 0.0, alpha, 0.2 * alpha)
        ex = jnp.exp(alpha)
        return dstv, ea, ex

    for half in range(2):
        chunk = 2 * tid + half
        base = chunk * ECH
        pltpu.sync_copy(src1.at[pl.ds(base, ECH)], srcb)
        pltpu.sync_copy(dst1.at[pl.ds(base, ECH)], dstb)
        ebase_f = lax.convert_element_type(base, jnp.float32)
        own = (chunk >= 16) == (c == 1)

        def _blk_compute(b, slot, own=own, base=base, ebase_f=ebase_f):
            for r in range(BLK):
                o = b * 400 + r * 80
                so = slot * 400 + r * 80
                pltpu.async_copy(s_hbm.at[srcb.at[pl.ds(o, 80)]],
                                 sblk.at[pl.ds(so, 80)], ssem).wait()
                pltpu.async_copy(d_hbm.at[dstb.at[pl.ds(o, 80)]],
                                 dblk.at[pl.ds(so, 80)], ssem).wait()
            for r in range(BLK):
                o = b * 400 + r * 80
                for g in range(5):
                    so = slot * 400 + r * 80 + g * 16
                    sg = sblk[pl.ds(so, 16)]
                    dg = dblk[pl.ds(so, 16)]
                    dstv, ea, ex = _edge_group(ebase_f, o + g * 16, sg, dg)
                    dst2a[slot * BLK + r, pl.ds(g * 16, 16)] = dstv
                    if emit_ea:
                        eablk[pl.ds(so, 16)] = ea
                    exblk[pl.ds(so, 16)] = ex
                pltpu.sync_copy(exblk.at[pl.ds(slot * 400 + r * 80, 80)],
                                den_sh.at[dst2a.at[slot * BLK + r]],
                                add=True)
            @pl.when(own)
            def _():
                pltpu.async_copy(exblk.at[pl.ds(slot * 400, 400)],
                                 ex_hbm.at[pl.ds(base + b * 400, 400)],
                                 easem).wait()
            if emit_ea:
                @pl.when(c == 0)
                def _():
                    pltpu.async_copy(
                        eablk.at[pl.ds(slot * 400, 400)],
                        ea_hbm.at[pl.ds(base + b * 400, 400)], easem).wait()

        def _ablk(b, _):
            _blk_compute(b, 0)
            return 0
        lax.fori_loop(0, NBLK, _ablk, 0)

    plsc.subcore_barrier()

    wid = c * 16 + tid
    base_b = wid * ECH
    pltpu.sync_copy(src1.at[pl.ds(base_b, ECH)], srcb)
    pltpu.sync_copy(dst1.at[pl.ds(base_b, ECH)], dstb)

    def _b_process(k, rbuf, slot):
        so = slot * 80
        gdesc = pltpu.async_copy(h_hbm.at[srcb.at[pl.ds(k * 80, 80)]],
                                 rbuf, gsem)
        pltpu.async_copy(ex_hbm.at[pl.ds(base_b + k * 80, 80)],
                         exrd.at[pl.ds(so, 80)], bsem).wait()
        pltpu.async_copy(den_sh.at[dstb.at[pl.ds(k * 80, 80)]],
                         dgblk.at[pl.ds(so, 80)], bsem).wait()
        gdesc.wait()
        for g in range(5):
            so = slot * 80 + g * 16
            dstv = dstb[pl.ds(k * 80 + g * 16, 16)]
            dst2b[slot, pl.ds(g * 16, 16)] = dstv
            ex = exrd[pl.ds(so, 16)]
            den = dgblk[pl.ds(so, 16)]
            coefv = ex / (den + 1e-16)
            for l in range(16):
                rr = g * 16 + l
                cl = _splat(coefv, l)
                for j in range(C // 16):
                    rbuf[rr, pl.ds(j * 16, 16)] = (
                        rbuf[rr, pl.ds(j * 16, 16)] * cl)
        pltpu.sync_copy(rbuf, out_sh.at[dst2b.at[slot]], add=True)

    def _bchunk(k, _):
        _b_process(k, rows0, 0)
        return 0
    lax.fori_loop(0, RB, _bchunk, 0)

    plsc.subcore_barrier()

    @pl.when(tid < 15)
    def _():
        pltpu.sync_copy(out_sh.at[pl.ds(tid * 640, 640)],
                        part_hbm.at[c, pl.ds(tid * 640, 640)])

    @pl.when(tid == 15)
    def _():
        pltpu.sync_copy(out_sh.at[pl.ds(15 * 640, N - 15 * 640)],
                        part_hbm.at[c, pl.ds(15 * 640, N - 15 * 640)])


def _make_sc_layer(emit_ea):
    out_type = [jax.ShapeDtypeStruct((2, N, C), jnp.float32),
                jax.ShapeDtypeStruct((E,), jnp.float32)]
    if emit_ea:
        out_type.append(jax.ShapeDtypeStruct((E,), jnp.float32))
    scratch = [
        pltpu.VMEM((ECH,), jnp.int32),
        pltpu.VMEM((ECH,), jnp.int32),
        pltpu.VMEM((80, C), jnp.float32),
        pltpu.VMEM((80, C), jnp.float32),
        pltpu.VMEM((800,), jnp.float32),
        pltpu.VMEM((800,), jnp.float32),
        pltpu.VMEM((800,), jnp.float32),
        pltpu.VMEM((160,), jnp.float32),
        pltpu.VMEM((2 * BLK, 80), jnp.int32),
        pltpu.VMEM((2, 80), jnp.int32),
        pltpu.VMEM((160,), jnp.float32),
        pltpu.VMEM((C,), jnp.float32),
        pltpu.VMEM((C,), jnp.float32),
        pltpu.VMEM((16,), jnp.float32),
        pltpu.VMEM_SHARED((NPAD,), jnp.float32),
        pltpu.VMEM_SHARED((NPAD, C), jnp.float32),
        pltpu.SemaphoreType.DMA,
        pltpu.SemaphoreType.DMA,
        pltpu.SemaphoreType.DMA,
        pltpu.SemaphoreType.DMA,
    ]
    if emit_ea:
        scratch.append(pltpu.VMEM((800,), jnp.float32))
    return pl.kernel(
        functools.partial(_sc_layer_body, emit_ea),
        out_type=out_type,
        mesh=plsc.VectorSubcoreMesh(**_MESH),
        scratch_types=scratch,
        compiler_params=pltpu.CompilerParams(needs_layout_passes=False),
    )


_sc_layer1 = _make_sc_layer(True)
_sc_layer2 = _make_sc_layer(False)


def kernel(x, edge_index, W1, a_s1, a_d1, We1, a_e1, b1,
           W2, a_s2, a_d2, We2, a_e2, b2, Wl, bl, Wew, bew):
    src1 = edge_index[0]
    dst1 = edge_index[1]
    ewp = jnp.concatenate([Wew.reshape(-1), bew.reshape(-1),
                           jnp.zeros((12,), jnp.float32)])
    p1 = jnp.concatenate([a_s1.reshape(1, -1), a_d1.reshape(1, -1),
                          jnp.zeros((6, D), jnp.float32)], axis=0)
    p2 = jnp.concatenate([a_s2.reshape(1, -1), a_d2.reshape(1, -1),
                          jnp.zeros((6, C), jnp.float32)], axis=0)
    wl_pad = jnp.zeros((C, 128), jnp.float32).at[:, 0].set(Wl[:, 0])
    bl_pad = jnp.zeros((1, 128), jnp.float32).at[0, 0].set(bl[0])

    h1 = _tc_in(x, W1)
    sdt1 = _tc_sd(p1, h1)
    part1, _, ea1 = _sc_layer1(src1, dst1, h1, sdt1[0], sdt1[1],
                               We1.reshape(-1), a_e1.reshape(-1), ewp)
    h2 = _tc_mid(part1[0], part1[1], b1.reshape(1, C), W2)
    sdt2 = _tc_sd(p2, h2)
    part2, _ = _sc_layer2(src1, dst1, h2, sdt2[0], sdt2[1],
                          We2.reshape(-1), a_e2.reshape(-1), ewp)
    outp = _tc_out(part2[0], part2[1], b2.reshape(1, C), wl_pad, bl_pad)
    return outp[:, :1], ea1.reshape(E, 1)


# speedup vs baseline: 17.7151x; 1.2222x over previous
import functools

import jax
import jax.numpy as jnp
from jax import lax
from jax.experimental import pallas as pl
from jax.experimental.pallas import tpu as pltpu
from jax.experimental.pallas import tpu_sc as plsc

N = 10000
E = 320000
C = 128
D = 128

NPAD = 10240
ECH = 10000
RB = 125
BLK = 5
NBLK = RB // BLK
BN = 2000

_ROW_B = 80 * C * 4
_SD_B = BLK * 5 * 16 * 4 * 2
_DEN_B = BLK * 5 * 16 * 4
_EA_B = BLK * 80 * 4

_MESH = dict(core_axis_name="c", subcore_axis_name="s", num_cores=2,
             num_subcores=16)


def _tc_in_body(x_ref, w_ref, h_ref):
    h_ref[...] = jnp.dot(x_ref[...], w_ref[...],
                         preferred_element_type=jnp.float32)


def _tc_in(x, w):
    return pl.pallas_call(
        _tc_in_body,
        grid=(N // BN,),
        in_specs=[pl.BlockSpec((BN, D), lambda i: (i, 0)),
                  pl.BlockSpec((D, C), lambda i: (0, 0))],
        out_specs=pl.BlockSpec((BN, C), lambda i: (i, 0)),
        out_shape=jax.ShapeDtypeStruct((N, C), jnp.float32),
    )(x, w)


def _tc_sd_body(p_ref, h_ref, sdt_ref):
    sdt_ref[...] = lax.dot_general(p_ref[...], h_ref[...],
                                   (((1,), (1,)), ((), ())),
                                   preferred_element_type=jnp.float32)


def _tc_sd(p, h):
    return pl.pallas_call(
        _tc_sd_body,
        in_specs=[pl.BlockSpec((8, C), lambda: (0, 0)),
                  pl.BlockSpec((N, C), lambda: (0, 0))],
        out_specs=pl.BlockSpec((8, N), lambda: (0, 0)),
        out_shape=jax.ShapeDtypeStruct((8, N), jnp.float32),
    )(p, h)


def _tc_mid_body(q0_ref, q1_ref, b_ref, w_ref, h_ref):
    o = jnp.maximum(q0_ref[...] + q1_ref[...] + b_ref[...], 0.0)
    h_ref[...] = jnp.dot(o, w_ref[...], preferred_element_type=jnp.float32)


def _tc_mid(q0, q1, b, w):
    return pl.pallas_call(
        _tc_mid_body,
        grid=(N // BN,),
        in_specs=[pl.BlockSpec((BN, C), lambda i: (i, 0)),
                  pl.BlockSpec((BN, C), lambda i: (i, 0)),
                  pl.BlockSpec((1, C), lambda i: (0, 0)),
                  pl.BlockSpec((C, C), lambda i: (0, 0))],
        out_specs=pl.BlockSpec((BN, C), lambda i: (i, 0)),
        out_shape=jax.ShapeDtypeStruct((N, C), jnp.float32),
    )(q0, q1, b, w)


def _tc_out_body(q0_ref, q1_ref, b_ref, wl_ref, bl_ref, out_ref):
    o = jnp.maximum(q0_ref[...] + q1_ref[...] + b_ref[...], 0.0)
    out_ref[...] = jnp.dot(o, wl_ref[...],
                           preferred_element_type=jnp.float32) + bl_ref[...]


def _tc_out(q0, q1, b, wl, bl):
    return pl.pallas_call(
        _tc_out_body,
        grid=(N // BN,),
        in_specs=[pl.BlockSpec((BN, C), lambda i: (i, 0)),
                  pl.BlockSpec((BN, C), lambda i: (i, 0)),
                  pl.BlockSpec((1, C), lambda i: (0, 0)),
                  pl.BlockSpec((C, 128), lambda i: (0, 0)),
                  pl.BlockSpec((1, 128), lambda i: (0, 0))],
        out_specs=pl.BlockSpec((BN, 128), lambda i: (i, 0)),
        out_shape=jax.ShapeDtypeStruct((N, 128), jnp.float32),
    )(q0, q1, b, wl, bl)


def _splat(v, lane):
    idx = jnp.full((16, 1), lane, dtype=jnp.int32)
    dn = lax.GatherDimensionNumbers(offset_dims=(), collapsed_slice_dims=(0,),
                                    start_index_map=(0,))
    return lax.gather(v, idx, dn, slice_sizes=(1,),
                      mode=lax.GatherScatterMode.PROMISE_IN_BOUNDS)


def _sc_layer_body(emit_ea, *refs):
    if emit_ea:
        (src1, dst1, h_hbm, s_hbm, d_hbm, we_hbm, ae_hbm, ewp_hbm,
         part_hbm, ea_hbm,
         srcb, dstb, rows0, rows1, sblk, dblk, exblk, dgblk, dst2a, dst2b,
         wev, aev, ewpv,
         den_sh, out_sh,
         gsem, ssem, bsem, easem, eablk) = refs
    else:
        (src1, dst1, h_hbm, s_hbm, d_hbm, we_hbm, ae_hbm, ewp_hbm,
         part_hbm,
         srcb, dstb, rows0, rows1, sblk, dblk, exblk, dgblk, dst2a, dst2b,
         wev, aev, ewpv,
         den_sh, out_sh,
         gsem, ssem, bsem, easem) = refs
        ea_hbm = eablk = None

    c = lax.axis_index("c")
    tid = lax.axis_index("s")

    def _ex_zero(i, _):
        exblk[pl.ds(i * 16, 16)] = jnp.zeros((16,), jnp.float32)
        return 0
    lax.fori_loop(0, 25, _ex_zero, 0)
    pltpu.sync_copy(exblk.at[pl.ds(0, 400)], den_sh.at[pl.ds(tid * 640, 400)])
    pltpu.sync_copy(exblk.at[pl.ds(0, 240)],
                    den_sh.at[pl.ds(tid * 640 + 400, 240)])

    def _r0_zero(i, _):
        for j in range(C // 16):
            rows0[i, pl.ds(j * 16, 16)] = jnp.zeros((16,), jnp.float32)
        return 0
    lax.fori_loop(0, 80, _r0_zero, 0)

    def _out_zero(k, _):
        pltpu.sync_copy(rows0, out_sh.at[pl.ds(tid * 640 + k * 80, 80)])
        return 0
    lax.fori_loop(0, 8, _out_zero, 0)

    pltpu.sync_copy(we_hbm, wev)
    pltpu.sync_copy(ae_hbm, aev)
    pltpu.sync_copy(ewp_hbm, ewpv)

    acc = wev[pl.ds(0, 16)] * aev[pl.ds(0, 16)]
    for j in range(1, C // 16):
        acc = acc + wev[pl.ds(j * 16, 16)] * aev[pl.ds(j * 16, 16)]
    ce = _splat(acc, 0)
    for l in range(1, 16):
        ce = ce + _splat(acc, l)

    ewv = ewpv[pl.ds(0, 16)]
    w0 = ewv[0]
    w1 = ewv[1]
    w2 = ewv[2]
    bew = ewv[3]

    lanes_f = lax.convert_element_type(
        lax.broadcasted_iota(jnp.int32, (16,), 0), jnp.float32)

    plsc.subcore_barrier()

    def _edge_group(ebase_f, off, sg, dg):
        srcv = srcb[pl.ds(off, 16)]
        dstv = dstb[pl.ds(off, 16)]
        eidf = ebase_f + lax.convert_element_type(off, jnp.float32) + lanes_f
        lin = (eidf * w0
               + lax.convert_element_type(srcv, jnp.float32) * w1
               + lax.convert_element_type(dstv, jnp.float32) * w2
               + bew)
        ea = 1.0 / (1.0 + jnp.exp(-lin))
        alpha = sg + dg + ce * ea
        alpha = jnp.where(alpha >= 0.0, alpha, 0.2 * alpha)
        ex = jnp.exp(alpha)
        return dstv, ea, ex

    for half in range(2):
        chunk = 2 * tid + half
        base = chunk * ECH
        pltpu.sync_copy(src1.at[pl.ds(base, ECH)], srcb)
        pltpu.sync_copy(dst1.at[pl.ds(base, ECH)], dstb)
        ebase_f = lax.convert_element_type(base, jnp.float32)
        own = (chunk >= 16) == (c == 1)

        def _blk_compute(b, slot, own=own, base=base, ebase_f=ebase_f):
            for r in range(BLK):
                o = b * 400 + r * 80
                so = slot * 400 + r * 80
                d1 = pltpu.async_copy(s_hbm.at[srcb.at[pl.ds(o, 80)]],
                                      sblk.at[pl.ds(so, 80)], ssem)
                d2 = pltpu.async_copy(d_hbm.at[dstb.at[pl.ds(o, 80)]],
                                      dblk.at[pl.ds(so, 80)], bsem)
                d1.wait()
                d2.wait()
            for r in range(BLK):
                o = b * 400 + r * 80
                for g in range(5):
                    so = slot * 400 + r * 80 + g * 16
                    sg = sblk[pl.ds(so, 16)]
                    dg = dblk[pl.ds(so, 16)]
                    dstv, ea, ex = _edge_group(ebase_f, o + g * 16, sg, dg)
                    dst2a[slot * BLK + r, pl.ds(g * 16, 16)] = dstv
                    if emit_ea:
                        eablk[pl.ds(so, 16)] = ea
                    exblk[pl.ds(so, 16)] = ex
                pltpu.sync_copy(exblk.at[pl.ds(slot * 400 + r * 80, 80)],
                                den_sh.at[dst2a.at[slot * BLK + r]],
                                add=True)
            if emit_ea:
                @pl.when(c == 0)
                def _():
                    pltpu.async_copy(
                        eablk.at[pl.ds(slot * 400, 400)],
                        ea_hbm.at[pl.ds(base + b * 400, 400)], easem).wait()

        def _ablk(b, _):
            _blk_compute(b, 0)
            return 0
        lax.fori_loop(0, NBLK, _ablk, 0)

    plsc.subcore_barrier()

    wid = c * 16 + tid
    base_b = wid * ECH
    pltpu.sync_copy(src1.at[pl.ds(base_b, ECH)], srcb)
    pltpu.sync_copy(dst1.at[pl.ds(base_b, ECH)], dstb)

    ebase_b = lax.convert_element_type(base_b, jnp.float32)

    def _b_process(k, rbuf, slot):
        so = slot * 80
        gdesc = pltpu.async_copy(h_hbm.at[srcb.at[pl.ds(k * 80, 80)]],
                                 rbuf, gsem)
        d1 = pltpu.async_copy(s_hbm.at[srcb.at[pl.ds(k * 80, 80)]],
                              sblk.at[pl.ds(so, 80)], bsem)
        d2 = pltpu.async_copy(d_hbm.at[dstb.at[pl.ds(k * 80, 80)]],
                              dblk.at[pl.ds(so, 80)], ssem)
        d3 = pltpu.async_copy(den_sh.at[dstb.at[pl.ds(k * 80, 80)]],
                              dgblk.at[pl.ds(so, 80)], easem)
        d1.wait()
        d2.wait()
        d3.wait()
        gdesc.wait()
        for g in range(5):
            so = slot * 80 + g * 16
            sg = sblk[pl.ds(so, 16)]
            dg = dblk[pl.ds(so, 16)]
            dstv, _, ex = _edge_group(ebase_b, k * 80 + g * 16, sg, dg)
            dst2b[slot, pl.ds(g * 16, 16)] = dstv
            den = dgblk[pl.ds(so, 16)]
            coefv = ex / (den + 1e-16)
            for l in range(16):
                rr = g * 16 + l
                cl = _splat(coefv, l)
                for j in range(C // 16):
                    rbuf[rr, pl.ds(j * 16, 16)] = (
                        rbuf[rr, pl.ds(j * 16, 16)] * cl)
        pltpu.sync_copy(rbuf, out_sh.at[dst2b.at[slot]], add=True)

    def _bchunk(k, _):
        _b_process(k, rows0, 0)
        return 0
    lax.fori_loop(0, RB, _bchunk, 0)

    plsc.subcore_barrier()

    @pl.when(tid < 15)
    def _():
        pltpu.sync_copy(out_sh.at[pl.ds(tid * 640, 640)],
                        part_hbm.at[c, pl.ds(tid * 640, 640)])

    @pl.when(tid == 15)
    def _():
        pltpu.sync_copy(out_sh.at[pl.ds(15 * 640, N - 15 * 640)],
                        part_hbm.at[c, pl.ds(15 * 640, N - 15 * 640)])


def _make_sc_layer(emit_ea):
    out_type = [jax.ShapeDtypeStruct((2, N, C), jnp.float32)]
    if emit_ea:
        out_type.append(jax.ShapeDtypeStruct((E,), jnp.float32))
    scratch = [
        pltpu.VMEM((ECH,), jnp.int32),
        pltpu.VMEM((ECH,), jnp.int32),
        pltpu.VMEM((80, C), jnp.float32),
        pltpu.VMEM((80, C), jnp.float32),
        pltpu.VMEM((800,), jnp.float32),
        pltpu.VMEM((800,), jnp.float32),
        pltpu.VMEM((800,), jnp.float32),
        pltpu.VMEM((160,), jnp.float32),
        pltpu.VMEM((2 * BLK, 80), jnp.int32),
        pltpu.VMEM((2, 80), jnp.int32),
        pltpu.VMEM((C,), jnp.float32),
        pltpu.VMEM((C,), jnp.float32),
        pltpu.VMEM((16,), jnp.float32),
        pltpu.VMEM_SHARED((NPAD,), jnp.float32),
        pltpu.VMEM_SHARED((NPAD, C), jnp.float32),
        pltpu.SemaphoreType.DMA,
        pltpu.SemaphoreType.DMA,
        pltpu.SemaphoreType.DMA,
        pltpu.SemaphoreType.DMA,
    ]
    if emit_ea:
        scratch.append(pltpu.VMEM((800,), jnp.float32))
    return pl.kernel(
        functools.partial(_sc_layer_body, emit_ea),
        out_type=out_type,
        mesh=plsc.VectorSubcoreMesh(**_MESH),
        scratch_types=scratch,
        compiler_params=pltpu.CompilerParams(needs_layout_passes=False),
    )


_sc_layer1 = _make_sc_layer(True)
_sc_layer2 = _make_sc_layer(False)


def kernel(x, edge_index, W1, a_s1, a_d1, We1, a_e1, b1,
           W2, a_s2, a_d2, We2, a_e2, b2, Wl, bl, Wew, bew):
    src1 = edge_index[0]
    dst1 = edge_index[1]
    ewp = jnp.concatenate([Wew.reshape(-1), bew.reshape(-1),
                           jnp.zeros((12,), jnp.float32)])
    p1 = jnp.concatenate([a_s1.reshape(1, -1), a_d1.reshape(1, -1),
                          jnp.zeros((6, D), jnp.float32)], axis=0)
    p2 = jnp.concatenate([a_s2.reshape(1, -1), a_d2.reshape(1, -1),
                          jnp.zeros((6, C), jnp.float32)], axis=0)
    wl_pad = jnp.zeros((C, 128), jnp.float32).at[:, 0].set(Wl[:, 0])
    bl_pad = jnp.zeros((1, 128), jnp.float32).at[0, 0].set(bl[0])

    h1 = _tc_in(x, W1)
    sdt1 = _tc_sd(p1, h1)
    part1, ea1 = _sc_layer1(src1, dst1, h1, sdt1[0], sdt1[1],
                            We1.reshape(-1), a_e1.reshape(-1), ewp)
    h2 = _tc_mid(part1[0], part1[1], b1.reshape(1, C), W2)
    sdt2 = _tc_sd(p2, h2)
    part2 = _sc_layer2(src1, dst1, h2, sdt2[0], sdt2[1],
                       We2.reshape(-1), a_e2.reshape(-1), ewp)
    if isinstance(part2, (list, tuple)):
        part2 = part2[0]
    outp = _tc_out(part2[0], part2[1], b2.reshape(1, C), wl_pad, bl_pad)
    return outp[:, :1], ea1.reshape(E, 1)


# speedup vs baseline: 26.2208x; 1.4801x over previous
import functools

import jax
import jax.numpy as jnp
from jax import lax
from jax.experimental import pallas as pl
from jax.experimental.pallas import tpu as pltpu
from jax.experimental.pallas import tpu_sc as plsc

N = 10000
E = 320000
C = 128
D = 128

NPAD = 10240
ECH = 10000
RB = 125
BLK = 5
NBLK = RB // BLK
BN = 2000

_ROW_B = 80 * C * 4
_SD_B = BLK * 5 * 16 * 4 * 2
_DEN_B = BLK * 5 * 16 * 4
_EA_B = BLK * 80 * 4

_MESH = dict(core_axis_name="c", subcore_axis_name="s", num_cores=2,
             num_subcores=16)


def _tc_in_body(x_ref, w_ref, h_ref):
    h_ref[...] = jnp.dot(x_ref[...], w_ref[...],
                         preferred_element_type=jnp.float32)


def _tc_in(x, w):
    return pl.pallas_call(
        _tc_in_body,
        grid=(N // BN,),
        in_specs=[pl.BlockSpec((BN, D), lambda i: (i, 0)),
                  pl.BlockSpec((D, C), lambda i: (0, 0))],
        out_specs=pl.BlockSpec((BN, C), lambda i: (i, 0)),
        out_shape=jax.ShapeDtypeStruct((N, C), jnp.float32),
    )(x, w)


def _tc_sd_body(p_ref, h_ref, sdt_ref):
    sdt_ref[...] = lax.dot_general(p_ref[...], h_ref[...],
                                   (((1,), (1,)), ((), ())),
                                   precision=lax.Precision.HIGHEST,
                                   preferred_element_type=jnp.float32)


def _tc_sd(p, h):
    return pl.pallas_call(
        _tc_sd_body,
        in_specs=[pl.BlockSpec((8, C), lambda: (0, 0)),
                  pl.BlockSpec((N, C), lambda: (0, 0))],
        out_specs=pl.BlockSpec((8, N), lambda: (0, 0)),
        out_shape=jax.ShapeDtypeStruct((8, N), jnp.float32),
    )(p, h)


def _tc_mid_body(q0_ref, q1_ref, b_ref, w_ref, h_ref):
    o = jnp.maximum(q0_ref[...] + q1_ref[...] + b_ref[...], 0.0)
    h_ref[...] = jnp.dot(o, w_ref[...], preferred_element_type=jnp.float32)


def _tc_mid(q0, q1, b, w):
    return pl.pallas_call(
        _tc_mid_body,
        grid=(N // BN,),
        in_specs=[pl.BlockSpec((BN, C), lambda i: (i, 0)),
                  pl.BlockSpec((BN, C), lambda i: (i, 0)),
                  pl.BlockSpec((1, C), lambda i: (0, 0)),
                  pl.BlockSpec((C, C), lambda i: (0, 0))],
        out_specs=pl.BlockSpec((BN, C), lambda i: (i, 0)),
        out_shape=jax.ShapeDtypeStruct((N, C), jnp.float32),
    )(q0, q1, b, w)


def _tc_out_body(q0_ref, q1_ref, b_ref, wl_ref, bl_ref, out_ref):
    o = jnp.maximum(q0_ref[...] + q1_ref[...] + b_ref[...], 0.0)
    out_ref[...] = jnp.dot(o, wl_ref[...],
                           preferred_element_type=jnp.float32) + bl_ref[...]


def _tc_out(q0, q1, b, wl, bl):
    return pl.pallas_call(
        _tc_out_body,
        grid=(N // BN,),
        in_specs=[pl.BlockSpec((BN, C), lambda i: (i, 0)),
                  pl.BlockSpec((BN, C), lambda i: (i, 0)),
                  pl.BlockSpec((1, C), lambda i: (0, 0)),
                  pl.BlockSpec((C, 128), lambda i: (0, 0)),
                  pl.BlockSpec((1, 128), lambda i: (0, 0))],
        out_specs=pl.BlockSpec((BN, 128), lambda i: (i, 0)),
        out_shape=jax.ShapeDtypeStruct((N, 128), jnp.float32),
    )(q0, q1, b, wl, bl)


def _splat(v, lane):
    idx = jnp.full((16, 1), lane, dtype=jnp.int32)
    dn = lax.GatherDimensionNumbers(offset_dims=(), collapsed_slice_dims=(0,),
                                    start_index_map=(0,))
    return lax.gather(v, idx, dn, slice_sizes=(1,),
                      mode=lax.GatherScatterMode.PROMISE_IN_BOUNDS)


def _sc_layer_body(emit_ea, *refs):
    nout = 2 if emit_ea else 1
    (src1, dst1, h_hbm, s_hbm, d_hbm, we_hbm, ae_hbm, ewp_hbm) = refs[:8]
    part_hbm = refs[8]
    ea_hbm = refs[9] if emit_ea else None
    sc = refs[8 + nout:]
    (srcb, dstb, rows0, rows1, sblk, dblk, exblk, dgblk, dst2a, dst2b,
     wev, aev, ewpv, den_sh, out_sh, easem) = sc[:16]
    asems = sc[16:26]
    bsems = sc[26:34]
    eablk = sc[34] if emit_ea else None

    c = lax.axis_index("c")
    tid = lax.axis_index("s")

    def _ex_zero(i, _):
        exblk[pl.ds(i * 16, 16)] = jnp.zeros((16,), jnp.float32)
        return 0
    lax.fori_loop(0, 25, _ex_zero, 0)
    pltpu.sync_copy(exblk.at[pl.ds(0, 400)], den_sh.at[pl.ds(tid * 640, 400)])
    pltpu.sync_copy(exblk.at[pl.ds(0, 240)],
                    den_sh.at[pl.ds(tid * 640 + 400, 240)])

    def _r0_zero(i, _):
        for j in range(C // 16):
            rows0[i, pl.ds(j * 16, 16)] = jnp.zeros((16,), jnp.float32)
        return 0
    lax.fori_loop(0, 80, _r0_zero, 0)

    def _out_zero(k, _):
        pltpu.sync_copy(rows0, out_sh.at[pl.ds(tid * 640 + k * 80, 80)])
        return 0
    lax.fori_loop(0, 8, _out_zero, 0)

    pltpu.sync_copy(we_hbm, wev)
    pltpu.sync_copy(ae_hbm, aev)
    pltpu.sync_copy(ewp_hbm, ewpv)

    acc = wev[pl.ds(0, 16)] * aev[pl.ds(0, 16)]
    for j in range(1, C // 16):
        acc = acc + wev[pl.ds(j * 16, 16)] * aev[pl.ds(j * 16, 16)]
    ce = _splat(acc, 0)
    for l in range(1, 16):
        ce = ce + _splat(acc, l)

    ewv = ewpv[pl.ds(0, 16)]
    w0 = ewv[0]
    w1 = ewv[1]
    w2 = ewv[2]
    bew = ewv[3]

    lanes_f = lax.convert_element_type(
        lax.broadcasted_iota(jnp.int32, (16,), 0), jnp.float32)

    plsc.subcore_barrier()

    def _edge_group(ebase_f, off, sg, dg):
        srcv = srcb[pl.ds(off, 16)]
        dstv = dstb[pl.ds(off, 16)]
        eidf = ebase_f + lax.convert_element_type(off, jnp.float32) + lanes_f
        lin = (eidf * w0
               + lax.convert_element_type(srcv, jnp.float32) * w1
               + lax.convert_element_type(dstv, jnp.float32) * w2
               + bew)
        ea = 1.0 / (1.0 + jnp.exp(-lin))
        alpha = sg + dg + ce * ea
        alpha = jnp.where(alpha >= 0.0, alpha, 0.2 * alpha)
        ex = jnp.exp(alpha)
        return dstv, ea, ex

    for half in range(2):
        chunk = 2 * tid + half
        base = chunk * ECH
        pltpu.sync_copy(src1.at[pl.ds(base, ECH)], srcb)
        pltpu.sync_copy(dst1.at[pl.ds(base, ECH)], dstb)
        ebase_f = lax.convert_element_type(base, jnp.float32)
        own = (chunk >= 16) == (c == 1)

        def _blk_compute(b, slot, own=own, base=base, ebase_f=ebase_f):
            descs = []
            for r in range(BLK):
                o = b * 400 + r * 80
                so = slot * 400 + r * 80
                descs.append(
                    pltpu.async_copy(s_hbm.at[srcb.at[pl.ds(o, 80)]],
                                     sblk.at[pl.ds(so, 80)], asems[2 * r]))
                descs.append(
                    pltpu.async_copy(d_hbm.at[dstb.at[pl.ds(o, 80)]],
                                     dblk.at[pl.ds(so, 80)],
                                     asems[2 * r + 1]))
            for r in range(BLK):
                o = b * 400 + r * 80
                descs[2 * r].wait()
                descs[2 * r + 1].wait()
                for g in range(5):
                    so = slot * 400 + r * 80 + g * 16
                    sg = sblk[pl.ds(so, 16)]
                    dg = dblk[pl.ds(so, 16)]
                    dstv, ea, ex = _edge_group(ebase_f, o + g * 16, sg, dg)
                    dst2a[slot * BLK + r, pl.ds(g * 16, 16)] = dstv
                    if emit_ea:
                        eablk[pl.ds(so, 16)] = ea
                    exblk[pl.ds(so, 16)] = ex
                pltpu.sync_copy(exblk.at[pl.ds(slot * 400 + r * 80, 80)],
                                den_sh.at[dst2a.at[slot * BLK + r]],
                                add=True)
            if emit_ea:
                @pl.when(c == 0)
                def _():
                    pltpu.async_copy(
                        eablk.at[pl.ds(slot * 400, 400)],
                        ea_hbm.at[pl.ds(base + b * 400, 400)], easem).wait()

        def _ablk(b, _):
            _blk_compute(b, 0)
            return 0
        lax.fori_loop(0, NBLK, _ablk, 0)

    plsc.subcore_barrier()

    wid = c * 16 + tid
    base_b = wid * ECH
    pltpu.sync_copy(src1.at[pl.ds(base_b, ECH)], srcb)
    pltpu.sync_copy(dst1.at[pl.ds(base_b, ECH)], dstb)

    ebase_b = lax.convert_element_type(base_b, jnp.float32)

    def _b_copies(k, rbuf, par):
        so = par * 80
        return (
            pltpu.make_async_copy(h_hbm.at[srcb.at[pl.ds(k * 80, 80)]],
                                  rbuf, bsems[par * 4]),
            pltpu.make_async_copy(s_hbm.at[srcb.at[pl.ds(k * 80, 80)]],
                                  sblk.at[pl.ds(so, 80)], bsems[par * 4 + 1]),
            pltpu.make_async_copy(d_hbm.at[dstb.at[pl.ds(k * 80, 80)]],
                                  dblk.at[pl.ds(so, 80)], bsems[par * 4 + 2]),
            pltpu.make_async_copy(den_sh.at[dstb.at[pl.ds(k * 80, 80)]],
                                  dgblk.at[pl.ds(so, 80)], bsems[par * 4 + 3]),
        )

    def _b_fire(k, rbuf, par):
        for cp in _b_copies(k, rbuf, par):
            cp.start()

    def _b_process(k, rbuf, par):
        slot = par
        so = slot * 80
        for cp in _b_copies(k, rbuf, par):
            cp.wait()
        for g in range(5):
            so = slot * 80 + g * 16
            sg = sblk[pl.ds(so, 16)]
            dg = dblk[pl.ds(so, 16)]
            dstv, _, ex = _edge_group(ebase_b, k * 80 + g * 16, sg, dg)
            dst2b[slot, pl.ds(g * 16, 16)] = dstv
            den = dgblk[pl.ds(so, 16)]
            coefv = ex / (den + 1e-16)
            for l in range(16):
                rr = g * 16 + l
                cl = _splat(coefv, l)
                for j in range(C // 16):
                    rbuf[rr, pl.ds(j * 16, 16)] = (
                        rbuf[rr, pl.ds(j * 16, 16)] * cl)
        pltpu.sync_copy(rbuf, out_sh.at[dst2b.at[slot]], add=True)

    _b_fire(0, rows0, 0)
    _b_fire(1, rows1, 1)

    def _bpair(i, _):
        k0 = 2 * i
        k1 = 2 * i + 1
        _b_process(k0, rows0, 0)
        _b_fire(k0 + 2, rows0, 0)
        _b_process(k1, rows1, 1)

        @pl.when(k1 + 2 < RB)
        def _():
            _b_fire(k1 + 2, rows1, 1)
        return 0
    lax.fori_loop(0, (RB - 1) // 2, _bpair, 0)
    _b_process(RB - 1, rows0, 0)

    plsc.subcore_barrier()

    @pl.when(tid < 15)
    def _():
        pltpu.sync_copy(out_sh.at[pl.ds(tid * 640, 640)],
                        part_hbm.at[c, pl.ds(tid * 640, 640)])

    @pl.when(tid == 15)
    def _():
        pltpu.sync_copy(out_sh.at[pl.ds(15 * 640, N - 15 * 640)],
                        part_hbm.at[c, pl.ds(15 * 640, N - 15 * 640)])


def _make_sc_layer(emit_ea):
    out_type = [jax.ShapeDtypeStruct((2, N, C), jnp.float32)]
    if emit_ea:
        out_type.append(jax.ShapeDtypeStruct((E,), jnp.float32))
    scratch = [
        pltpu.VMEM((ECH,), jnp.int32),
        pltpu.VMEM((ECH,), jnp.int32),
        pltpu.VMEM((80, C), jnp.float32),
        pltpu.VMEM((80, C), jnp.float32),
        pltpu.VMEM((800,), jnp.float32),
        pltpu.VMEM((800,), jnp.float32),
        pltpu.VMEM((800,), jnp.float32),
        pltpu.VMEM((160,), jnp.float32),
        pltpu.VMEM((2 * BLK, 80), jnp.int32),
        pltpu.VMEM((2, 80), jnp.int32),
        pltpu.VMEM((C,), jnp.float32),
        pltpu.VMEM((C,), jnp.float32),
        pltpu.VMEM((16,), jnp.float32),
        pltpu.VMEM_SHARED((NPAD,), jnp.float32),
        pltpu.VMEM_SHARED((NPAD, C), jnp.float32),
        pltpu.SemaphoreType.DMA,
    ] + [pltpu.SemaphoreType.DMA] * 10 \
      + [pltpu.SemaphoreType.DMA] * 8
    if emit_ea:
        scratch.append(pltpu.VMEM((800,), jnp.float32))
    return pl.kernel(
        functools.partial(_sc_layer_body, emit_ea),
        out_type=out_type,
        mesh=plsc.VectorSubcoreMesh(**_MESH),
        scratch_types=scratch,
        compiler_params=pltpu.CompilerParams(needs_layout_passes=False),
    )


_sc_layer1 = _make_sc_layer(True)
_sc_layer2 = _make_sc_layer(False)


def kernel(x, edge_index, W1, a_s1, a_d1, We1, a_e1, b1,
           W2, a_s2, a_d2, We2, a_e2, b2, Wl, bl, Wew, bew):
    src1 = edge_index[0]
    dst1 = edge_index[1]
    ewp = jnp.concatenate([Wew.reshape(-1), bew.reshape(-1),
                           jnp.zeros((12,), jnp.float32)])
    p1 = jnp.concatenate([a_s1.reshape(1, -1), a_d1.reshape(1, -1),
                          jnp.zeros((6, D), jnp.float32)], axis=0)
    p2 = jnp.concatenate([a_s2.reshape(1, -1), a_d2.reshape(1, -1),
                          jnp.zeros((6, C), jnp.float32)], axis=0)
    wl_pad = jnp.zeros((C, 128), jnp.float32).at[:, 0].set(Wl[:, 0])
    bl_pad = jnp.zeros((1, 128), jnp.float32).at[0, 0].set(bl[0])

    h1 = _tc_in(x, W1)
    sdt1 = _tc_sd(p1, h1)
    part1, ea1 = _sc_layer1(src1, dst1, h1, sdt1[0], sdt1[1],
                            We1.reshape(-1), a_e1.reshape(-1), ewp)
    h2 = _tc_mid(part1[0], part1[1], b1.reshape(1, C), W2)
    sdt2 = _tc_sd(p2, h2)
    part2 = _sc_layer2(src1, dst1, h2, sdt2[0], sdt2[1],
                       We2.reshape(-1), a_e2.reshape(-1), ewp)
    if isinstance(part2, (list, tuple)):
        part2 = part2[0]
    outp = _tc_out(part2[0], part2[1], b2.reshape(1, C), wl_pad, bl_pad)
    return outp[:, :1], ea1.reshape(E, 1)


# speedup vs baseline: 26.2286x; 1.0003x over previous
import functools

import jax
import jax.numpy as jnp
from jax import lax
from jax.experimental import pallas as pl
from jax.experimental.pallas import tpu as pltpu
from jax.experimental.pallas import tpu_sc as plsc

N = 10000
E = 320000
C = 128
D = 128

NPAD = 10240
ECH = 10000
RB = 125
BLK = 5
NBLK = RB // BLK
BN = 2000

_ROW_B = 80 * C * 4
_SD_B = BLK * 5 * 16 * 4 * 2
_DEN_B = BLK * 5 * 16 * 4
_EA_B = BLK * 80 * 4

_MESH = dict(core_axis_name="c", subcore_axis_name="s", num_cores=2,
             num_subcores=16)


def _tc_in_body(x_ref, w_ref, h_ref):
    h_ref[...] = jnp.dot(x_ref[...], w_ref[...],
                         preferred_element_type=jnp.float32)


def _tc_in(x, w):
    return pl.pallas_call(
        _tc_in_body,
        grid=(N // BN,),
        in_specs=[pl.BlockSpec((BN, D), lambda i: (i, 0)),
                  pl.BlockSpec((D, C), lambda i: (0, 0))],
        out_specs=pl.BlockSpec((BN, C), lambda i: (i, 0)),
        out_shape=jax.ShapeDtypeStruct((N, C), jnp.float32),
    )(x, w)


def _tc_sd_body(p_ref, h_ref, sdt_ref):
    sdt_ref[...] = lax.dot_general(p_ref[...], h_ref[...],
                                   (((1,), (1,)), ((), ())),
                                   precision=lax.Precision.HIGHEST,
                                   preferred_element_type=jnp.float32)


def _tc_sd(p, h):
    return pl.pallas_call(
        _tc_sd_body,
        in_specs=[pl.BlockSpec((8, C), lambda: (0, 0)),
                  pl.BlockSpec((N, C), lambda: (0, 0))],
        out_specs=pl.BlockSpec((8, N), lambda: (0, 0)),
        out_shape=jax.ShapeDtypeStruct((8, N), jnp.float32),
    )(p, h)


def _tc_mid_body(q0_ref, q1_ref, b_ref, w_ref, h_ref):
    o = jnp.maximum(q0_ref[...] + q1_ref[...] + b_ref[...], 0.0)
    h_ref[...] = jnp.dot(o, w_ref[...], preferred_element_type=jnp.float32)


def _tc_mid(q0, q1, b, w):
    return pl.pallas_call(
        _tc_mid_body,
        grid=(N // BN,),
        in_specs=[pl.BlockSpec((BN, C), lambda i: (i, 0)),
                  pl.BlockSpec((BN, C), lambda i: (i, 0)),
                  pl.BlockSpec((1, C), lambda i: (0, 0)),
                  pl.BlockSpec((C, C), lambda i: (0, 0))],
        out_specs=pl.BlockSpec((BN, C), lambda i: (i, 0)),
        out_shape=jax.ShapeDtypeStruct((N, C), jnp.float32),
    )(q0, q1, b, w)


def _tc_out_body(q0_ref, q1_ref, b_ref, wl_ref, bl_ref, out_ref):
    o = jnp.maximum(q0_ref[...] + q1_ref[...] + b_ref[...], 0.0)
    out_ref[...] = jnp.dot(o, wl_ref[...],
                           preferred_element_type=jnp.float32) + bl_ref[...]


def _tc_out(q0, q1, b, wl, bl):
    return pl.pallas_call(
        _tc_out_body,
        grid=(N // BN,),
        in_specs=[pl.BlockSpec((BN, C), lambda i: (i, 0)),
                  pl.BlockSpec((BN, C), lambda i: (i, 0)),
                  pl.BlockSpec((1, C), lambda i: (0, 0)),
                  pl.BlockSpec((C, 128), lambda i: (0, 0)),
                  pl.BlockSpec((1, 128), lambda i: (0, 0))],
        out_specs=pl.BlockSpec((BN, 128), lambda i: (i, 0)),
        out_shape=jax.ShapeDtypeStruct((N, 128), jnp.float32),
    )(q0, q1, b, wl, bl)


def _splat(v, lane):
    idx = jnp.full((16, 1), lane, dtype=jnp.int32)
    dn = lax.GatherDimensionNumbers(offset_dims=(), collapsed_slice_dims=(0,),
                                    start_index_map=(0,))
    return lax.gather(v, idx, dn, slice_sizes=(1,),
                      mode=lax.GatherScatterMode.PROMISE_IN_BOUNDS)


def _sc_layer_body(emit_ea, *refs):
    nout = 2 if emit_ea else 1
    (src1, dst1, h_hbm, s_hbm, d_hbm, we_hbm, ae_hbm, ewp_hbm) = refs[:8]
    part_hbm = refs[8]
    ea_hbm = refs[9] if emit_ea else None
    sc = refs[8 + nout:]
    (srcb, dstb, rows0, rows1, sblk, dblk, exblk, dgblk, dst2a, dst2b,
     wev, aev, ewpv, den_sh, out_sh, easem) = sc[:16]
    asems = sc[16:26]
    bsems = sc[26:34]
    eablk = sc[34] if emit_ea else None

    c = lax.axis_index("c")
    tid = lax.axis_index("s")

    def _ex_zero(i, _):
        exblk[pl.ds(i * 16, 16)] = jnp.zeros((16,), jnp.float32)
        return 0
    lax.fori_loop(0, 25, _ex_zero, 0)
    pltpu.sync_copy(exblk.at[pl.ds(0, 400)], den_sh.at[pl.ds(tid * 640, 400)])
    pltpu.sync_copy(exblk.at[pl.ds(0, 240)],
                    den_sh.at[pl.ds(tid * 640 + 400, 240)])

    def _r0_zero(i, _):
        for j in range(C // 16):
            rows0[i, pl.ds(j * 16, 16)] = jnp.zeros((16,), jnp.float32)
        return 0
    lax.fori_loop(0, 80, _r0_zero, 0)

    def _out_zero(k, _):
        pltpu.sync_copy(rows0, out_sh.at[pl.ds(tid * 640 + k * 80, 80)])
        return 0
    lax.fori_loop(0, 8, _out_zero, 0)

    pltpu.sync_copy(we_hbm, wev)
    pltpu.sync_copy(ae_hbm, aev)
    pltpu.sync_copy(ewp_hbm, ewpv)

    acc = wev[pl.ds(0, 16)] * aev[pl.ds(0, 16)]
    for j in range(1, C // 16):
        acc = acc + wev[pl.ds(j * 16, 16)] * aev[pl.ds(j * 16, 16)]
    ce = _splat(acc, 0)
    for l in range(1, 16):
        ce = ce + _splat(acc, l)

    ewv = ewpv[pl.ds(0, 16)]
    w0 = ewv[0]
    w1 = ewv[1]
    w2 = ewv[2]
    bew = ewv[3]

    lanes_f = lax.convert_element_type(
        lax.broadcasted_iota(jnp.int32, (16,), 0), jnp.float32)

    plsc.subcore_barrier()

    def _edge_group(ebase_f, off, sg, dg):
        srcv = srcb[pl.ds(off, 16)]
        dstv = dstb[pl.ds(off, 16)]
        eidf = ebase_f + lax.convert_element_type(off, jnp.float32) + lanes_f
        lin = (eidf * w0
               + lax.convert_element_type(srcv, jnp.float32) * w1
               + lax.convert_element_type(dstv, jnp.float32) * w2
               + bew)
        ea = 1.0 / (1.0 + jnp.exp(-lin))
        alpha = sg + dg + ce * ea
        alpha = jnp.where(alpha >= 0.0, alpha, 0.2 * alpha)
        ex = jnp.exp(alpha)
        return dstv, ea, ex

    for half in range(2):
        chunk = 2 * tid + half
        base = chunk * ECH
        pltpu.sync_copy(src1.at[pl.ds(base, ECH)], srcb)
        pltpu.sync_copy(dst1.at[pl.ds(base, ECH)], dstb)
        ebase_f = lax.convert_element_type(base, jnp.float32)
        own = (chunk >= 16) == (c == 1)

        def _blk_compute(b, slot, own=own, base=base, ebase_f=ebase_f):
            descs = []
            for r in range(BLK):
                o = b * 400 + r * 80
                so = slot * 400 + r * 80
                descs.append(
                    pltpu.async_copy(s_hbm.at[srcb.at[pl.ds(o, 80)]],
                                     sblk.at[pl.ds(so, 80)], asems[2 * r]))
                descs.append(
                    pltpu.async_copy(d_hbm.at[dstb.at[pl.ds(o, 80)]],
                                     dblk.at[pl.ds(so, 80)],
                                     asems[2 * r + 1]))
            for r in range(BLK):
                o = b * 400 + r * 80
                descs[2 * r].wait()
                descs[2 * r + 1].wait()
                for g in range(5):
                    so = slot * 400 + r * 80 + g * 16
                    sg = sblk[pl.ds(so, 16)]
                    dg = dblk[pl.ds(so, 16)]
                    dstv, ea, ex = _edge_group(ebase_f, o + g * 16, sg, dg)
                    dst2a[slot * BLK + r, pl.ds(g * 16, 16)] = dstv
                    if emit_ea:
                        eablk[pl.ds(so, 16)] = ea
                    exblk[pl.ds(so, 16)] = ex
                pltpu.sync_copy(exblk.at[pl.ds(slot * 400 + r * 80, 80)],
                                den_sh.at[dst2a.at[slot * BLK + r]],
                                add=True)
            if emit_ea:
                @pl.when(c == 0)
                def _():
                    pltpu.async_copy(
                        eablk.at[pl.ds(slot * 400, 400)],
                        ea_hbm.at[pl.ds(base + b * 400, 400)], easem).wait()

        def _ablk(b, _):
            _blk_compute(b, 0)
            return 0
        lax.fori_loop(0, NBLK, _ablk, 0)

    plsc.subcore_barrier()

    wid = c * 16 + tid
    base_b = wid * ECH
    pltpu.sync_copy(src1.at[pl.ds(base_b, ECH)], srcb)
    pltpu.sync_copy(dst1.at[pl.ds(base_b, ECH)], dstb)

    ebase_b = lax.convert_element_type(base_b, jnp.float32)

    def _b_copies(k, rbuf, par):
        so = par * 80
        return (
            pltpu.make_async_copy(h_hbm.at[srcb.at[pl.ds(k * 80, 80)]],
                                  rbuf, bsems[par * 4]),
            pltpu.make_async_copy(s_hbm.at[srcb.at[pl.ds(k * 80, 80)]],
                                  sblk.at[pl.ds(so, 80)], bsems[par * 4 + 1]),
            pltpu.make_async_copy(d_hbm.at[dstb.at[pl.ds(k * 80, 80)]],
                                  dblk.at[pl.ds(so, 80)], bsems[par * 4 + 2]),
            pltpu.make_async_copy(den_sh.at[dstb.at[pl.ds(k * 80, 80)]],
                                  dgblk.at[pl.ds(so, 80)], bsems[par * 4 + 3]),
        )

    def _b_fire(k, rbuf, par):
        for cp in _b_copies(k, rbuf, par):
            cp.start()

    def _b_process(k, rbuf, par):
        slot = par
        so = slot * 80
        for cp in _b_copies(k, rbuf, par):
            cp.wait()
        for g in range(5):
            so = slot * 80 + g * 16
            sg = sblk[pl.ds(so, 16)]
            dg = dblk[pl.ds(so, 16)]
            dstv, _, ex = _edge_group(ebase_b, k * 80 + g * 16, sg, dg)
            dst2b[slot, pl.ds(g * 16, 16)] = dstv
            den = dgblk[pl.ds(so, 16)]
            coefv = ex / (den + 1e-16)
            for l in range(16):
                rr = g * 16 + l
                cl = _splat(coefv, l)
                for j in range(C // 16):
                    rbuf[rr, pl.ds(j * 16, 16)] = (
                        rbuf[rr, pl.ds(j * 16, 16)] * cl)
        pltpu.sync_copy(rbuf, out_sh.at[dst2b.at[slot]], add=True)

    _b_fire(0, rows0, 0)
    _b_fire(1, rows1, 1)

    def _bpair(i, _):
        k0 = 2 * i
        k1 = 2 * i + 1
        _b_process(k0, rows0, 0)
        _b_fire(k0 + 2, rows0, 0)
        _b_process(k1, rows1, 1)

        @pl.when(k1 + 2 < RB)
        def _():
            _b_fire(k1 + 2, rows1, 1)
        return 0
    lax.fori_loop(0, (RB - 1) // 2, _bpair, 0)
    _b_process(RB - 1, rows0, 0)

    plsc.subcore_barrier()

    @pl.when(tid < 15)
    def _():
        pltpu.sync_copy(out_sh.at[pl.ds(tid * 640, 640)],
                        part_hbm.at[c, pl.ds(tid * 640, 640)])

    @pl.when(tid == 15)
    def _():
        pltpu.sync_copy(out_sh.at[pl.ds(15 * 640, N - 15 * 640)],
                        part_hbm.at[c, pl.ds(15 * 640, N - 15 * 640)])


def _make_sc_layer(emit_ea):
    out_type = [jax.ShapeDtypeStruct((2, N, C), jnp.float32)]
    if emit_ea:
        out_type.append(jax.ShapeDtypeStruct((E,), jnp.float32))
    scratch = [
        pltpu.VMEM((ECH,), jnp.int32),
        pltpu.VMEM((ECH,), jnp.int32),
        pltpu.VMEM((80, C), jnp.float32),
        pltpu.VMEM((80, C), jnp.float32),
        pltpu.VMEM((800,), jnp.float32),
        pltpu.VMEM((800,), jnp.float32),
        pltpu.VMEM((800,), jnp.float32),
        pltpu.VMEM((160,), jnp.float32),
        pltpu.VMEM((2 * BLK, 80), jnp.int32),
        pltpu.VMEM((2, 80), jnp.int32),
        pltpu.VMEM((C,), jnp.float32),
        pltpu.VMEM((C,), jnp.float32),
        pltpu.VMEM((16,), jnp.float32),
        pltpu.VMEM_SHARED((NPAD,), jnp.float32),
        pltpu.VMEM_SHARED((NPAD, C), jnp.float32),
        pltpu.SemaphoreType.DMA,
    ] + [pltpu.SemaphoreType.DMA] * 10 \
      + [pltpu.SemaphoreType.DMA] * 8
    if emit_ea:
        scratch.append(pltpu.VMEM((800,), jnp.float32))
    return pl.kernel(
        functools.partial(_sc_layer_body, emit_ea),
        out_type=out_type,
        mesh=plsc.VectorSubcoreMesh(**_MESH),
        scratch_types=scratch,
        compiler_params=pltpu.CompilerParams(needs_layout_passes=False),
    )


_sc_cache = {}


def _sc_layer(emit_ea):
    if emit_ea not in _sc_cache:
        _sc_cache[emit_ea] = _make_sc_layer(emit_ea)
    return _sc_cache[emit_ea]


def kernel(x, edge_index, W1, a_s1, a_d1, We1, a_e1, b1,
           W2, a_s2, a_d2, We2, a_e2, b2, Wl, bl, Wew, bew):
    src1 = edge_index[0]
    dst1 = edge_index[1]
    ewp = jnp.concatenate([Wew.reshape(-1), bew.reshape(-1),
                           jnp.zeros((12,), jnp.float32)])
    p1 = jnp.concatenate([a_s1.reshape(1, -1), a_d1.reshape(1, -1),
                          jnp.zeros((6, D), jnp.float32)], axis=0)
    p2 = jnp.concatenate([a_s2.reshape(1, -1), a_d2.reshape(1, -1),
                          jnp.zeros((6, C), jnp.float32)], axis=0)
    wl_pad = jnp.zeros((C, 128), jnp.float32).at[:, 0].set(Wl[:, 0])
    bl_pad = jnp.zeros((1, 128), jnp.float32).at[0, 0].set(bl[0])

    h1 = _tc_in(x, W1)
    sdt1 = _tc_sd(p1, h1)
    part1, ea1 = _sc_layer(True)(src1, dst1, h1, sdt1[0], sdt1[1],
                                 We1.reshape(-1), a_e1.reshape(-1), ewp)
    h2 = _tc_mid(part1[0], part1[1], b1.reshape(1, C), W2)
    sdt2 = _tc_sd(p2, h2)
    part2 = _sc_layer(False)(src1, dst1, h2, sdt2[0], sdt2[1],
                             We2.reshape(-1), a_e2.reshape(-1), ewp)
    if isinstance(part2, (list, tuple)):
        part2 = part2[0]
    outp = _tc_out(part2[0], part2[1], b2.reshape(1, C), wl_pad, bl_pad)
    return outp[:, :1], ea1.reshape(E, 1)
